# Initial kernel scaffold; baseline (speedup 1.0000x reference)
import jax
import jax.numpy as jnp
from jax.experimental import pallas as pl


def kernel(qk, v, rotations):
    raise NotImplementedError("write your pallas kernel here")


# speedup vs baseline: 3.2930x; 3.2930x over previous
import functools
import jax
import jax.numpy as jnp
from jax import lax
from jax.experimental import pallas as pl
from jax.experimental.pallas import tpu as pltpu
from jax.experimental.pallas import tpu_sc as plsc

B, S, D = 16, 2048, 128
H = 8
NBK = 32
CH = 64
QB = 128
NQB = S // QB
GRP = 128
NGRP = S // GRP
SELF_VAL = -50000.0

_HIGH = lax.Precision.HIGHEST


def _hash_dest_body(qk_ref, rot_ref, dest_ref):
    qk = qk_ref[0]
    rot = rot_ref[...]
    rotated = lax.dot_general(qk, rot, (((1,), (0,)), ((), ())),
                              preferred_element_type=jnp.float32,
                              precision=lax.Precision.DEFAULT)
    iota32 = lax.broadcasted_iota(jnp.int32, (S, NBK), 1)
    oh_bf, lt_f, oh_f = [], [], []
    for h in range(H):
        rh = rotated[:, h * 16:(h + 1) * 16]
        cc = jnp.concatenate([rh, -rh], axis=1)
        m = jnp.max(cc, axis=1, keepdims=True)
        bidx = jnp.min(jnp.where(cc == m, iota32, NBK), axis=1,
                       keepdims=True)
        oh = bidx == iota32
        oh_bf.append(oh.astype(jnp.bfloat16))
        oh_f.append(oh.astype(jnp.float32))
        lt_f.append((bidx < iota32).astype(jnp.float32))
    OH = jnp.concatenate(oh_bf, axis=1)
    LT = jnp.concatenate(lt_f, axis=1)
    offs = jnp.sum(LT, axis=0, keepdims=True)

    r_i = lax.broadcasted_iota(jnp.int32, (GRP, GRP), 0)
    c_i = lax.broadcasted_iota(jnp.int32, (GRP, GRP), 1)
    Ls = (r_i > c_i).astype(jnp.bfloat16)
    base = jnp.zeros((1, H * NBK), jnp.float32)
    parts = []
    for g in range(NGRP):
        blk = OH[g * GRP:(g + 1) * GRP]
        cumg = lax.dot_general(Ls, blk, (((1,), (0,)), ((), ())),
                               preferred_element_type=jnp.float32)
        parts.append(cumg + base)
        base = base + jnp.sum(blk.astype(jnp.float32), axis=0, keepdims=True)
    RANK = jnp.concatenate(parts, axis=0)
    destf = RANK + offs
    cols = []
    for h in range(H):
        sel = destf[:, h * NBK:(h + 1) * NBK] * oh_f[h]
        cols.append(jnp.sum(sel, axis=1, keepdims=True))
    dest = jnp.concatenate(cols, axis=1).astype(jnp.int32)
    dest_ref[0] = jnp.transpose(dest, (1, 0))


def _hash_dest(qk, rot):
    return pl.pallas_call(
        _hash_dest_body,
        grid=(B,),
        in_specs=[
            pl.BlockSpec((1, S, D), lambda b: (b, 0, 0)),
            pl.BlockSpec((D, H * 16), lambda b: (0, 0)),
        ],
        out_specs=pl.BlockSpec((1, H, S), lambda b: (b, 0, 0)),
        out_shape=jax.ShapeDtypeStruct((B, H, S), jnp.int32),
    )(qk, rot)


_NCHUNK = S // 128


def _make_scatter_kernel():
    mesh = plsc.VectorSubcoreMesh(core_axis_name="c", subcore_axis_name="s")

    @functools.partial(
        pl.kernel,
        mesh=mesh,
        out_type=(
            jax.ShapeDtypeStruct((B, H, S, D), jnp.float32),
            jax.ShapeDtypeStruct((B, H, S, D), jnp.float32),
            jax.ShapeDtypeStruct((B, H, S), jnp.int32),
        ),
        scratch_types=[
            pltpu.VMEM((128,), jnp.int32),
            pltpu.VMEM((128, D), jnp.float32),
            pltpu.VMEM((128, D), jnp.float32),
            pltpu.VMEM((S,), jnp.int32),
            pltpu.SemaphoreType.DMA,
            pltpu.SemaphoreType.DMA,
        ],
        compiler_params=pltpu.CompilerParams(needs_layout_passes=False),
    )
    def scatter_k(qk_hbm, v_hbm, dest_hbm, sqk_hbm, sv_hbm, st_hbm,
                  idx_v, rq_v, rv_v, st_v, sem1, sem2):
        wid = lax.axis_index("s") * 2 + lax.axis_index("c")
        for p in range(4):
            g = wid * 4 + p
            b = g // H
            h = g % H
            for c in range(_NCHUNK):
                cs = c * 128
                pltpu.sync_copy(dest_hbm.at[b, h, pl.ds(cs, 128)], idx_v)
                pltpu.sync_copy(qk_hbm.at[b, pl.ds(cs, 128)], rq_v)
                pltpu.sync_copy(v_hbm.at[b, pl.ds(cs, 128)], rv_v)
                cp1 = pltpu.async_copy(rq_v, sqk_hbm.at[b, h].at[idx_v], sem1)
                cp2 = pltpu.async_copy(rv_v, sv_hbm.at[b, h].at[idx_v], sem2)
                for k in range(8):
                    idx16 = idx_v[pl.ds(k * 16, 16)]
                    vals = (cs + k * 16) + lax.iota(jnp.int32, 16)
                    plsc.store_scatter(st_v, [idx16], vals)
                cp1.wait()
                cp2.wait()
            pltpu.sync_copy(st_v, st_hbm.at[b, h])

    return scatter_k


def _attn_body(sqk_ref, sv_ref, st_ref, hk_ref, hv_ref, ht_ref,
               so_ref, slog_ref, kx_ref, vx_ref, tx_ref):
    kx_ref[0:CH] = hk_ref[0, 0]
    kx_ref[CH:CH + S] = sqk_ref[0, 0]
    vx_ref[0:CH] = hv_ref[0, 0]
    vx_ref[CH:CH + S] = sv_ref[0, 0]
    tx_ref[0:CH] = ht_ref[0, 0]
    tx_ref[CH:CH + S] = st_ref[0, 0]

    r_i = lax.broadcasted_iota(jnp.int32, (QB, QB + CH), 0)
    c_i = lax.broadcasted_iota(jnp.int32, (QB, QB + CH), 1)
    qchunk = (r_i // CH) * CH
    band = (c_i >= qchunk) & (c_i < qchunk + 2 * CH)
    scale = D ** -0.5

    def step(i, _):
        q = kx_ref[pl.ds(CH + i * QB, QB), :]
        kw = kx_ref[pl.ds(i * QB, QB + CH), :]
        vw = vx_ref[pl.ds(i * QB, QB + CH), :]
        tq = tx_ref[pl.ds(CH + i * QB, QB)]
        tk = jnp.transpose(tx_ref[pl.ds(i * QB, QB + CH)], (1, 0))
        ssq = jnp.sum(kw * kw, axis=1, keepdims=True)
        norm = jnp.maximum(jnp.sqrt(ssq), 1e-12)
        kn = kw / norm
        dots = lax.dot_general(q, kn, (((1,), (1,)), ((), ())),
                               preferred_element_type=jnp.float32,
                               precision=_HIGH) * scale
        dots = jnp.where(tq == tk, SELF_VAL, dots)
        dots = jnp.where(band, dots, -1e30)
        m = jnp.max(dots, axis=1, keepdims=True)
        p = jnp.exp(dots - m)
        l = jnp.sum(p, axis=1, keepdims=True)
        lse = m + jnp.log(l)
        o = lax.dot_general(p / l, vw, (((1,), (0,)), ((), ())),
                            preferred_element_type=jnp.float32,
                            precision=_HIGH)
        so_ref[0, 0, pl.ds(i * QB, QB), :] = o
        slog_ref[0, 0, pl.ds(i * QB, QB), :] = lse
        return 0

    lax.fori_loop(0, NQB, step, 0)


def _attention(sqk, sv, st2, halo_k, halo_v, halo_t2):
    return pl.pallas_call(
        _attn_body,
        grid=(B, H),
        in_specs=[
            pl.BlockSpec((1, 1, S, D), lambda b, h: (b, h, 0, 0)),
            pl.BlockSpec((1, 1, S, D), lambda b, h: (b, h, 0, 0)),
            pl.BlockSpec((1, 1, S, 1), lambda b, h: (b, h, 0, 0)),
            pl.BlockSpec((1, 1, CH, D), lambda b, h: (b, h, 0, 0)),
            pl.BlockSpec((1, 1, CH, D), lambda b, h: (b, h, 0, 0)),
            pl.BlockSpec((1, 1, CH, 1), lambda b, h: (b, h, 0, 0)),
        ],
        out_specs=[
            pl.BlockSpec((1, 1, S, D), lambda b, h: (b, h, 0, 0)),
            pl.BlockSpec((1, 1, S, 1), lambda b, h: (b, h, 0, 0)),
        ],
        out_shape=[
            jax.ShapeDtypeStruct((B, H, S, D), jnp.float32),
            jax.ShapeDtypeStruct((B, H, S, 1), jnp.float32),
        ],
        scratch_shapes=[
            pltpu.VMEM((S + CH, D), jnp.float32),
            pltpu.VMEM((S + CH, D), jnp.float32),
            pltpu.VMEM((S + CH, 1), jnp.int32),
        ],
    )(sqk, sv, st2, halo_k, halo_v, halo_t2)


def _make_gather_kernel():
    mesh = plsc.VectorSubcoreMesh(core_axis_name="c", subcore_axis_name="s")

    @functools.partial(
        pl.kernel,
        mesh=mesh,
        out_type=(
            jax.ShapeDtypeStruct((B, H, S, D), jnp.float32),
            jax.ShapeDtypeStruct((B, H, S), jnp.float32),
        ),
        scratch_types=[
            pltpu.VMEM((128,), jnp.int32),
            pltpu.VMEM((128, D), jnp.float32),
            pltpu.VMEM((S,), jnp.float32),
            pltpu.VMEM((128,), jnp.float32),
            pltpu.SemaphoreType.DMA,
        ],
        compiler_params=pltpu.CompilerParams(needs_layout_passes=False),
    )
    def gather_k(so_hbm, slog_hbm, dest_hbm, ot_hbm, lt_hbm,
                 idx_v, rows_v, sl_v, lg_v, sem):
        wid = lax.axis_index("s") * 2 + lax.axis_index("c")
        for p in range(4):
            g = wid * 4 + p
            b = g // H
            h = g % H
            pltpu.sync_copy(slog_hbm.at[b, h], sl_v)
            for c in range(_NCHUNK):
                cs = c * 128
                pltpu.sync_copy(dest_hbm.at[b, h, pl.ds(cs, 128)], idx_v)
                cp = pltpu.async_copy(so_hbm.at[b, h].at[idx_v], rows_v, sem)
                for k in range(8):
                    idx16 = idx_v[pl.ds(k * 16, 16)]
                    lg_v[pl.ds(k * 16, 16)] = plsc.load_gather(sl_v, [idx16])
                pltpu.sync_copy(lg_v, lt_hbm.at[b, h, pl.ds(cs, 128)])
                cp.wait()
                pltpu.sync_copy(rows_v, ot_hbm.at[b, h, pl.ds(cs, 128)])

    return gather_k


_TS = 512


def _combine_body(ot_ref, lt_ref, out_ref):
    lg = lt_ref[0]
    m = jnp.max(lg, axis=0, keepdims=True)
    p = jnp.exp(lg - m)
    ssum = jnp.sum(p, axis=0, keepdims=True)
    w = p / ssum
    wt = jnp.transpose(w, (1, 0))
    acc = jnp.zeros((_TS, D), jnp.float32)
    for h in range(H):
        acc = acc + ot_ref[0, h] * wt[:, h:h + 1]
    out_ref[0] = acc


def _combine(o_tok, log_tok):
    return pl.pallas_call(
        _combine_body,
        grid=(B, S // _TS),
        in_specs=[
            pl.BlockSpec((1, H, _TS, D), lambda b, t: (b, 0, t, 0)),
            pl.BlockSpec((1, H, _TS), lambda b, t: (b, 0, t)),
        ],
        out_specs=pl.BlockSpec((1, _TS, D), lambda b, t: (b, t, 0)),
        out_shape=jax.ShapeDtypeStruct((B, S, D), jnp.float32),
    )(o_tok, log_tok)


_make_scatter_kernel = functools.cache(_make_scatter_kernel)
_make_gather_kernel = functools.cache(_make_gather_kernel)


@jax.jit
def kernel(qk, v, rotations):
    rot = rotations.reshape(D, H * 16)
    dest = _hash_dest(qk, rot)
    sqk, sv, st = _make_scatter_kernel()(qk, v, dest)
    halo_k = jnp.roll(sqk[:, :, S - CH:, :], 1, axis=1)
    halo_v = jnp.roll(sv[:, :, S - CH:, :], 1, axis=1)
    halo_t = jnp.roll(st[:, :, S - CH:], 1, axis=1)
    so, slog = _attention(sqk, sv, st.reshape(B, H, S, 1),
                          halo_k, halo_v, halo_t.reshape(B, H, CH, 1))
    o_tok, log_tok = _make_gather_kernel()(so, slog.reshape(B, H, S), dest)
    return _combine(o_tok, log_tok)


# speedup vs baseline: 4.8703x; 1.4790x over previous
import functools
import jax
import jax.numpy as jnp
from jax import lax
from jax.experimental import pallas as pl
from jax.experimental.pallas import tpu as pltpu
from jax.experimental.pallas import tpu_sc as plsc

B, S, D = 16, 2048, 128
H = 8
NBK = 32
CH = 64
QB = 128
NQB = S // QB
GRP = 128
NGRP = S // GRP
SELF_VAL = -50000.0

_HIGH = lax.Precision.HIGHEST


def _hash_dest_body(qk_ref, rot_ref, dest_ref):
    qk = qk_ref[0]
    rot = rot_ref[...]
    rotated = lax.dot_general(qk, rot, (((1,), (0,)), ((), ())),
                              preferred_element_type=jnp.float32,
                              precision=lax.Precision.DEFAULT)
    iota32 = lax.broadcasted_iota(jnp.int32, (S, NBK), 1)
    oh_bf, lt_f, oh_f = [], [], []
    for h in range(H):
        rh = rotated[:, h * 16:(h + 1) * 16]
        cc = jnp.concatenate([rh, -rh], axis=1)
        m = jnp.max(cc, axis=1, keepdims=True)
        bidx = jnp.min(jnp.where(cc == m, iota32, NBK), axis=1,
                       keepdims=True)
        oh = bidx == iota32
        oh_bf.append(oh.astype(jnp.bfloat16))
        oh_f.append(oh.astype(jnp.float32))
        lt_f.append((bidx < iota32).astype(jnp.float32))
    OH = jnp.concatenate(oh_bf, axis=1)
    LT = jnp.concatenate(lt_f, axis=1)
    offs = jnp.sum(LT, axis=0, keepdims=True)

    r_i = lax.broadcasted_iota(jnp.int32, (GRP, GRP), 0)
    c_i = lax.broadcasted_iota(jnp.int32, (GRP, GRP), 1)
    Ls = (r_i > c_i).astype(jnp.bfloat16)
    base = jnp.zeros((1, H * NBK), jnp.float32)
    parts = []
    for g in range(NGRP):
        blk = OH[g * GRP:(g + 1) * GRP]
        cumg = lax.dot_general(Ls, blk, (((1,), (0,)), ((), ())),
                               preferred_element_type=jnp.float32)
        parts.append(cumg + base)
        base = base + jnp.sum(blk.astype(jnp.float32), axis=0, keepdims=True)
    RANK = jnp.concatenate(parts, axis=0)
    destf = RANK + offs
    cols = []
    for h in range(H):
        sel = destf[:, h * NBK:(h + 1) * NBK] * oh_f[h]
        cols.append(jnp.sum(sel, axis=1, keepdims=True))
    dest = jnp.concatenate(cols, axis=1).astype(jnp.int32)
    dest_ref[0] = jnp.transpose(dest, (1, 0))


def _hash_dest(qk, rot):
    return pl.pallas_call(
        _hash_dest_body,
        grid=(B,),
        in_specs=[
            pl.BlockSpec((1, S, D), lambda b: (b, 0, 0)),
            pl.BlockSpec((D, H * 16), lambda b: (0, 0)),
        ],
        out_specs=pl.BlockSpec((1, H, S), lambda b: (b, 0, 0)),
        out_shape=jax.ShapeDtypeStruct((B, H, S), jnp.int32),
    )(qk, rot)


_NCHUNK = S // 128


def _make_scatter_kernel():
    mesh = plsc.VectorSubcoreMesh(core_axis_name="c", subcore_axis_name="s")

    @functools.partial(
        pl.kernel,
        mesh=mesh,
        out_type=(
            jax.ShapeDtypeStruct((B, H, S, D), jnp.float32),
            jax.ShapeDtypeStruct((B, H, S, D), jnp.float32),
            jax.ShapeDtypeStruct((B, H, S), jnp.int32),
        ),
        scratch_types=[
            pltpu.VMEM((128,), jnp.int32),
            pltpu.VMEM((128, D), jnp.float32),
            pltpu.VMEM((128, D), jnp.float32),
            pltpu.VMEM((S,), jnp.int32),
            pltpu.SemaphoreType.DMA,
            pltpu.SemaphoreType.DMA,
        ],
        compiler_params=pltpu.CompilerParams(needs_layout_passes=False),
    )
    def scatter_k(qk_hbm, v_hbm, dest_hbm, sqk_hbm, sv_hbm, st_hbm,
                  idx_v, rq_v, rv_v, st_v, sem1, sem2):
        wid = lax.axis_index("s") * 2 + lax.axis_index("c")
        for p in range(4):
            g = wid * 4 + p
            b = g // H
            h = g % H
            for c in range(_NCHUNK):
                cs = c * 128
                pltpu.sync_copy(dest_hbm.at[b, h, pl.ds(cs, 128)], idx_v)
                pltpu.sync_copy(qk_hbm.at[b, pl.ds(cs, 128)], rq_v)
                pltpu.sync_copy(v_hbm.at[b, pl.ds(cs, 128)], rv_v)
                cp1 = pltpu.async_copy(rq_v, sqk_hbm.at[b, h].at[idx_v], sem1)
                cp2 = pltpu.async_copy(rv_v, sv_hbm.at[b, h].at[idx_v], sem2)
                for k in range(8):
                    idx16 = idx_v[pl.ds(k * 16, 16)]
                    vals = (cs + k * 16) + lax.iota(jnp.int32, 16)
                    plsc.store_scatter(st_v, [idx16], vals)
                cp1.wait()
                cp2.wait()
            pltpu.sync_copy(st_v, st_hbm.at[b, h])

    return scatter_k


def _attn_body(sqk_ref, sv_ref, st_ref, hk_ref, hv_ref, ht_ref,
               so_ref, slog_ref, tx_ref):
    tx_ref[0:CH] = ht_ref[0, 0]
    tx_ref[CH:CH + S] = st_ref[0, 0]

    r_i = lax.broadcasted_iota(jnp.int32, (QB, QB + CH), 0)
    c_i = lax.broadcasted_iota(jnp.int32, (QB, QB + CH), 1)
    qchunk = (r_i // CH) * CH
    band = (c_i >= qchunk) & (c_i < qchunk + 2 * CH)
    scale = D ** -0.5

    def block(i, q, kw, vw, tq, tk):
        ssq = jnp.sum(kw * kw, axis=1, keepdims=True)
        rnorm = 1.0 / jnp.maximum(jnp.sqrt(ssq), 1e-12)
        kn = kw * rnorm
        dots = lax.dot_general(q, kn, (((1,), (1,)), ((), ())),
                               preferred_element_type=jnp.float32) * scale
        dots = jnp.where(tq == tk, SELF_VAL, dots)
        dots = jnp.where(band, dots, -1e30)
        m = jnp.max(dots, axis=1, keepdims=True)
        p = jnp.exp(dots - m)
        l = jnp.sum(p, axis=1, keepdims=True)
        lse = m + jnp.log(l)
        o = lax.dot_general(p, vw, (((1,), (0,)), ((), ())),
                            preferred_element_type=jnp.float32)
        so_ref[0, 0, pl.ds(i * QB, QB), :] = o * (1.0 / l)
        slog_ref[0, 0, pl.ds(i * QB, QB), :] = lse

    q0 = sqk_ref[0, 0, 0:QB, :]
    kw0 = jnp.concatenate([hk_ref[0, 0], sqk_ref[0, 0, 0:QB, :]], axis=0)
    vw0 = jnp.concatenate([hv_ref[0, 0], sv_ref[0, 0, 0:QB, :]], axis=0)
    tq0 = tx_ref[CH:CH + QB]
    tk0 = jnp.transpose(tx_ref[0:QB + CH], (1, 0))
    block(0, q0, kw0, vw0, tq0, tk0)

    def step(i, _):
        q = sqk_ref[0, 0, pl.ds(i * QB, QB), :]
        kw = sqk_ref[0, 0, pl.ds(i * QB - CH, QB + CH), :]
        vw = sv_ref[0, 0, pl.ds(i * QB - CH, QB + CH), :]
        tq = tx_ref[pl.ds(CH + i * QB, QB)]
        tk = jnp.transpose(tx_ref[pl.ds(i * QB, QB + CH)], (1, 0))
        block(i, q, kw, vw, tq, tk)
        return 0

    lax.fori_loop(1, NQB, step, 0)


def _attention(sqk, sv, st2, halo_k, halo_v, halo_t2):
    return pl.pallas_call(
        _attn_body,
        grid=(B, H),
        in_specs=[
            pl.BlockSpec((1, 1, S, D), lambda b, h: (b, h, 0, 0)),
            pl.BlockSpec((1, 1, S, D), lambda b, h: (b, h, 0, 0)),
            pl.BlockSpec((1, 1, S, 1), lambda b, h: (b, h, 0, 0)),
            pl.BlockSpec((1, 1, CH, D), lambda b, h: (b, h, 0, 0)),
            pl.BlockSpec((1, 1, CH, D), lambda b, h: (b, h, 0, 0)),
            pl.BlockSpec((1, 1, CH, 1), lambda b, h: (b, h, 0, 0)),
        ],
        out_specs=[
            pl.BlockSpec((1, 1, S, D), lambda b, h: (b, h, 0, 0)),
            pl.BlockSpec((1, 1, S, 1), lambda b, h: (b, h, 0, 0)),
        ],
        out_shape=[
            jax.ShapeDtypeStruct((B, H, S, D), jnp.float32),
            jax.ShapeDtypeStruct((B, H, S, 1), jnp.float32),
        ],
        scratch_shapes=[
            pltpu.VMEM((S + CH, 1), jnp.int32),
        ],
    )(sqk, sv, st2, halo_k, halo_v, halo_t2)


def _make_gather_kernel():
    mesh = plsc.VectorSubcoreMesh(core_axis_name="c", subcore_axis_name="s")

    @functools.partial(
        pl.kernel,
        mesh=mesh,
        out_type=(
            jax.ShapeDtypeStruct((B, H, S, D), jnp.float32),
            jax.ShapeDtypeStruct((B, H, S), jnp.float32),
        ),
        scratch_types=[
            pltpu.VMEM((128,), jnp.int32),
            pltpu.VMEM((128, D), jnp.float32),
            pltpu.VMEM((S,), jnp.float32),
            pltpu.VMEM((128,), jnp.float32),
            pltpu.SemaphoreType.DMA,
        ],
        compiler_params=pltpu.CompilerParams(needs_layout_passes=False),
    )
    def gather_k(so_hbm, slog_hbm, dest_hbm, ot_hbm, lt_hbm,
                 idx_v, rows_v, sl_v, lg_v, sem):
        wid = lax.axis_index("s") * 2 + lax.axis_index("c")
        for p in range(4):
            g = wid * 4 + p
            b = g // H
            h = g % H
            pltpu.sync_copy(slog_hbm.at[b, h], sl_v)
            for c in range(_NCHUNK):
                cs = c * 128
                pltpu.sync_copy(dest_hbm.at[b, h, pl.ds(cs, 128)], idx_v)
                cp = pltpu.async_copy(so_hbm.at[b, h].at[idx_v], rows_v, sem)
                for k in range(8):
                    idx16 = idx_v[pl.ds(k * 16, 16)]
                    lg_v[pl.ds(k * 16, 16)] = plsc.load_gather(sl_v, [idx16])
                pltpu.sync_copy(lg_v, lt_hbm.at[b, h, pl.ds(cs, 128)])
                cp.wait()
                pltpu.sync_copy(rows_v, ot_hbm.at[b, h, pl.ds(cs, 128)])

    return gather_k


_TS = 512


def _combine_body(ot_ref, lt_ref, out_ref):
    lg = lt_ref[0]
    m = jnp.max(lg, axis=0, keepdims=True)
    p = jnp.exp(lg - m)
    ssum = jnp.sum(p, axis=0, keepdims=True)
    w = p / ssum
    wt = jnp.transpose(w, (1, 0))
    acc = jnp.zeros((_TS, D), jnp.float32)
    for h in range(H):
        acc = acc + ot_ref[0, h] * wt[:, h:h + 1]
    out_ref[0] = acc


def _combine(o_tok, log_tok):
    return pl.pallas_call(
        _combine_body,
        grid=(B, S // _TS),
        in_specs=[
            pl.BlockSpec((1, H, _TS, D), lambda b, t: (b, 0, t, 0)),
            pl.BlockSpec((1, H, _TS), lambda b, t: (b, 0, t)),
        ],
        out_specs=pl.BlockSpec((1, _TS, D), lambda b, t: (b, t, 0)),
        out_shape=jax.ShapeDtypeStruct((B, S, D), jnp.float32),
    )(o_tok, log_tok)


_make_scatter_kernel = functools.cache(_make_scatter_kernel)
_make_gather_kernel = functools.cache(_make_gather_kernel)


@jax.jit
def kernel(qk, v, rotations):
    rot = rotations.reshape(D, H * 16)
    dest = _hash_dest(qk, rot)
    sqk, sv, st = _make_scatter_kernel()(qk, v, dest)
    halo_k = jnp.roll(sqk[:, :, S - CH:, :], 1, axis=1)
    halo_v = jnp.roll(sv[:, :, S - CH:, :], 1, axis=1)
    halo_t = jnp.roll(st[:, :, S - CH:], 1, axis=1)
    so, slog = _attention(sqk, sv, st.reshape(B, H, S, 1),
                          halo_k, halo_v, halo_t.reshape(B, H, CH, 1))
    o_tok, log_tok = _make_gather_kernel()(so, slog.reshape(B, H, S), dest)
    return _combine(o_tok, log_tok)


# speedup vs baseline: 5.8850x; 1.2083x over previous
import functools
import jax
import jax.numpy as jnp
from jax import lax
from jax.experimental import pallas as pl
from jax.experimental.pallas import tpu as pltpu
from jax.experimental.pallas import tpu_sc as plsc

B, S, D = 16, 2048, 128
H = 8
NBK = 32
CH = 64
QB = 256
NQB = S // QB
GRP = 128
NGRP = S // GRP
SELF_VAL = -50000.0

_HIGH = lax.Precision.HIGHEST


def _hash_dest_body(qk_ref, rot_ref, dest_ref):
    qk = qk_ref[0]
    rot = rot_ref[...]
    rotated = lax.dot_general(qk, rot, (((1,), (0,)), ((), ())),
                              preferred_element_type=jnp.float32,
                              precision=lax.Precision.DEFAULT)
    iota32 = lax.broadcasted_iota(jnp.int32, (S, NBK), 1)
    oh_bf, lt_f, oh_f = [], [], []
    for h in range(H):
        rh = rotated[:, h * 16:(h + 1) * 16]
        cc = jnp.concatenate([rh, -rh], axis=1)
        m = jnp.max(cc, axis=1, keepdims=True)
        bidx = jnp.min(jnp.where(cc == m, iota32, NBK), axis=1,
                       keepdims=True)
        oh = bidx == iota32
        oh_bf.append(oh.astype(jnp.bfloat16))
        oh_f.append(oh.astype(jnp.float32))
        lt_f.append((bidx < iota32).astype(jnp.float32))
    OH = jnp.concatenate(oh_bf, axis=1)
    LT = jnp.concatenate(lt_f, axis=1)
    offs = jnp.sum(LT, axis=0, keepdims=True)

    r_i = lax.broadcasted_iota(jnp.int32, (GRP, GRP), 0)
    c_i = lax.broadcasted_iota(jnp.int32, (GRP, GRP), 1)
    Ls = (r_i > c_i).astype(jnp.bfloat16)
    base = jnp.zeros((1, H * NBK), jnp.float32)
    parts = []
    for g in range(NGRP):
        blk = OH[g * GRP:(g + 1) * GRP]
        cumg = lax.dot_general(Ls, blk, (((1,), (0,)), ((), ())),
                               preferred_element_type=jnp.float32)
        parts.append(cumg + base)
        base = base + jnp.sum(blk.astype(jnp.float32), axis=0, keepdims=True)
    RANK = jnp.concatenate(parts, axis=0)
    destf = RANK + offs
    cols = []
    for h in range(H):
        sel = destf[:, h * NBK:(h + 1) * NBK] * oh_f[h]
        cols.append(jnp.sum(sel, axis=1, keepdims=True))
    dest = jnp.concatenate(cols, axis=1).astype(jnp.int32)
    dest_ref[0] = jnp.transpose(dest, (1, 0))


def _hash_dest(qk, rot):
    return pl.pallas_call(
        _hash_dest_body,
        grid=(B,),
        in_specs=[
            pl.BlockSpec((1, S, D), lambda b: (b, 0, 0)),
            pl.BlockSpec((D, H * 16), lambda b: (0, 0)),
        ],
        out_specs=pl.BlockSpec((1, H, S), lambda b: (b, 0, 0)),
        out_shape=jax.ShapeDtypeStruct((B, H, S), jnp.int32),
    )(qk, rot)


_NCHUNK = S // 128


def _make_scatter_kernel():
    mesh = plsc.VectorSubcoreMesh(core_axis_name="c", subcore_axis_name="s")

    @functools.partial(
        pl.kernel,
        mesh=mesh,
        out_type=(
            jax.ShapeDtypeStruct((B, H, S, D), jnp.float32),
            jax.ShapeDtypeStruct((B, H, S, D), jnp.float32),
            jax.ShapeDtypeStruct((B, H, S), jnp.int32),
        ),
        scratch_types=[
            pltpu.VMEM((128,), jnp.int32),
            pltpu.VMEM((128, D), jnp.float32),
            pltpu.VMEM((128, D), jnp.float32),
            pltpu.VMEM((S,), jnp.int32),
            pltpu.SemaphoreType.DMA,
            pltpu.SemaphoreType.DMA,
        ],
        compiler_params=pltpu.CompilerParams(needs_layout_passes=False),
    )
    def scatter_k(qk_hbm, v_hbm, dest_hbm, sqk_hbm, sv_hbm, st_hbm,
                  idx_v, rq_v, rv_v, st_v, sem1, sem2):
        wid = lax.axis_index("s") * 2 + lax.axis_index("c")
        for p in range(4):
            g = wid * 4 + p
            b = g // H
            h = g % H
            for c in range(_NCHUNK):
                cs = c * 128
                pltpu.sync_copy(dest_hbm.at[b, h, pl.ds(cs, 128)], idx_v)
                pltpu.sync_copy(qk_hbm.at[b, pl.ds(cs, 128)], rq_v)
                pltpu.sync_copy(v_hbm.at[b, pl.ds(cs, 128)], rv_v)
                cp1 = pltpu.async_copy(rq_v, sqk_hbm.at[b, h].at[idx_v], sem1)
                cp2 = pltpu.async_copy(rv_v, sv_hbm.at[b, h].at[idx_v], sem2)
                for k in range(8):
                    idx16 = idx_v[pl.ds(k * 16, 16)]
                    vals = (cs + k * 16) + lax.iota(jnp.int32, 16)
                    plsc.store_scatter(st_v, [idx16], vals)
                cp1.wait()
                cp2.wait()
            pltpu.sync_copy(st_v, st_hbm.at[b, h])

    return scatter_k


def _attn_body(sqk_ref, sv_ref, st_ref, hk_ref, hv_ref, ht_ref,
               so_ref, slog_ref, tx_ref):
    tx_ref[0:CH] = ht_ref[0, 0]
    tx_ref[CH:CH + S] = st_ref[0, 0]

    r_i = lax.broadcasted_iota(jnp.int32, (QB, QB + CH), 0)
    c_i = lax.broadcasted_iota(jnp.int32, (QB, QB + CH), 1)
    qchunk = (r_i // CH) * CH
    band = (c_i >= qchunk) & (c_i < qchunk + 2 * CH)
    scale = D ** -0.5

    def block(i, q, kw, vw, tq, tk):
        ssq = jnp.sum(kw * kw, axis=1, keepdims=True)
        rnorm = 1.0 / jnp.maximum(jnp.sqrt(ssq), 1e-12)
        kn = kw * rnorm
        dots = lax.dot_general(q, kn, (((1,), (1,)), ((), ())),
                               preferred_element_type=jnp.float32) * scale
        dots = jnp.where(tq == tk, SELF_VAL, dots)
        dots = jnp.where(band, dots, -1e30)
        m = jnp.max(dots, axis=1, keepdims=True)
        p = jnp.exp(dots - m)
        l = jnp.sum(p, axis=1, keepdims=True)
        lse = m + jnp.log(l)
        o = lax.dot_general(p, vw, (((1,), (0,)), ((), ())),
                            preferred_element_type=jnp.float32)
        so_ref[0, 0, pl.ds(i * QB, QB), :] = o * (1.0 / l)
        slog_ref[0, 0, pl.ds(i * QB, QB), :] = lse

    q0 = sqk_ref[0, 0, 0:QB, :]
    kw0 = jnp.concatenate([hk_ref[0, 0], sqk_ref[0, 0, 0:QB, :]], axis=0)
    vw0 = jnp.concatenate([hv_ref[0, 0], sv_ref[0, 0, 0:QB, :]], axis=0)
    tq0 = tx_ref[CH:CH + QB]
    tk0 = jnp.transpose(tx_ref[0:QB + CH], (1, 0))
    block(0, q0, kw0, vw0, tq0, tk0)

    def step(i, _):
        q = sqk_ref[0, 0, pl.ds(i * QB, QB), :]
        kw = sqk_ref[0, 0, pl.ds(i * QB - CH, QB + CH), :]
        vw = sv_ref[0, 0, pl.ds(i * QB - CH, QB + CH), :]
        tq = tx_ref[pl.ds(CH + i * QB, QB)]
        tk = jnp.transpose(tx_ref[pl.ds(i * QB, QB + CH)], (1, 0))
        block(i, q, kw, vw, tq, tk)
        return 0

    lax.fori_loop(1, NQB, step, 0)


def _attention(sqk, sv, st2, halo_k, halo_v, halo_t2):
    return pl.pallas_call(
        _attn_body,
        grid=(B, H),
        in_specs=[
            pl.BlockSpec((1, 1, S, D), lambda b, h: (b, h, 0, 0)),
            pl.BlockSpec((1, 1, S, D), lambda b, h: (b, h, 0, 0)),
            pl.BlockSpec((1, 1, S, 1), lambda b, h: (b, h, 0, 0)),
            pl.BlockSpec((1, 1, CH, D), lambda b, h: (b, h, 0, 0)),
            pl.BlockSpec((1, 1, CH, D), lambda b, h: (b, h, 0, 0)),
            pl.BlockSpec((1, 1, CH, 1), lambda b, h: (b, h, 0, 0)),
        ],
        out_specs=[
            pl.BlockSpec((1, 1, S, D), lambda b, h: (b, h, 0, 0)),
            pl.BlockSpec((1, 1, S, 1), lambda b, h: (b, h, 0, 0)),
        ],
        out_shape=[
            jax.ShapeDtypeStruct((B, H, S, D), jnp.float32),
            jax.ShapeDtypeStruct((B, H, S, 1), jnp.float32),
        ],
        scratch_shapes=[
            pltpu.VMEM((S + CH, 1), jnp.int32),
        ],
    )(sqk, sv, st2, halo_k, halo_v, halo_t2)


def _make_gather_kernel():
    mesh = plsc.VectorSubcoreMesh(core_axis_name="c", subcore_axis_name="s")

    @functools.partial(
        pl.kernel,
        mesh=mesh,
        out_type=(
            jax.ShapeDtypeStruct((B, H, S, D), jnp.float32),
            jax.ShapeDtypeStruct((B, H, S), jnp.float32),
        ),
        scratch_types=[
            pltpu.VMEM((128,), jnp.int32),
            pltpu.VMEM((128, D), jnp.float32),
            pltpu.VMEM((S,), jnp.float32),
            pltpu.VMEM((128,), jnp.float32),
            pltpu.SemaphoreType.DMA,
        ],
        compiler_params=pltpu.CompilerParams(needs_layout_passes=False),
    )
    def gather_k(so_hbm, slog_hbm, dest_hbm, ot_hbm, lt_hbm,
                 idx_v, rows_v, sl_v, lg_v, sem):
        wid = lax.axis_index("s") * 2 + lax.axis_index("c")
        for p in range(4):
            g = wid * 4 + p
            b = g // H
            h = g % H
            pltpu.sync_copy(slog_hbm.at[b, h], sl_v)
            for c in range(_NCHUNK):
                cs = c * 128
                pltpu.sync_copy(dest_hbm.at[b, h, pl.ds(cs, 128)], idx_v)
                cp = pltpu.async_copy(so_hbm.at[b, h].at[idx_v], rows_v, sem)
                for k in range(8):
                    idx16 = idx_v[pl.ds(k * 16, 16)]
                    lg_v[pl.ds(k * 16, 16)] = plsc.load_gather(sl_v, [idx16])
                pltpu.sync_copy(lg_v, lt_hbm.at[b, h, pl.ds(cs, 128)])
                cp.wait()
                pltpu.sync_copy(rows_v, ot_hbm.at[b, h, pl.ds(cs, 128)])

    return gather_k


_TS = 512


def _combine_body(ot_ref, lt_ref, out_ref):
    lg = lt_ref[0]
    m = jnp.max(lg, axis=0, keepdims=True)
    p = jnp.exp(lg - m)
    ssum = jnp.sum(p, axis=0, keepdims=True)
    w = p / ssum
    wt = jnp.transpose(w, (1, 0))
    acc = jnp.zeros((_TS, D), jnp.float32)
    for h in range(H):
        acc = acc + ot_ref[0, h] * wt[:, h:h + 1]
    out_ref[0] = acc


def _combine(o_tok, log_tok):
    return pl.pallas_call(
        _combine_body,
        grid=(B, S // _TS),
        in_specs=[
            pl.BlockSpec((1, H, _TS, D), lambda b, t: (b, 0, t, 0)),
            pl.BlockSpec((1, H, _TS), lambda b, t: (b, 0, t)),
        ],
        out_specs=pl.BlockSpec((1, _TS, D), lambda b, t: (b, t, 0)),
        out_shape=jax.ShapeDtypeStruct((B, S, D), jnp.float32),
    )(o_tok, log_tok)


_make_scatter_kernel = functools.cache(_make_scatter_kernel)
_make_gather_kernel = functools.cache(_make_gather_kernel)


@jax.jit
def kernel(qk, v, rotations):
    rot = rotations.reshape(D, H * 16)
    dest = _hash_dest(qk, rot)
    sqk, sv, st = _make_scatter_kernel()(qk, v, dest)
    halo_k = jnp.roll(sqk[:, :, S - CH:, :], 1, axis=1)
    halo_v = jnp.roll(sv[:, :, S - CH:, :], 1, axis=1)
    halo_t = jnp.roll(st[:, :, S - CH:], 1, axis=1)
    so, slog = _attention(sqk, sv, st.reshape(B, H, S, 1),
                          halo_k, halo_v, halo_t.reshape(B, H, CH, 1))
    o_tok, log_tok = _make_gather_kernel()(so, slog.reshape(B, H, S), dest)
    return _combine(o_tok, log_tok)


# speedup vs baseline: 6.3038x; 1.0712x over previous
import functools
import jax
import jax.numpy as jnp
from jax import lax
from jax.experimental import pallas as pl
from jax.experimental.pallas import tpu as pltpu
from jax.experimental.pallas import tpu_sc as plsc

B, S, D = 16, 2048, 128
H = 8
NBK = 32
CH = 64
QB = 256
NQB = S // QB
GRP = 128
NGRP = S // GRP
SELF_VAL = -50000.0

_HIGH = lax.Precision.HIGHEST


def _hash_dest_body(qk_ref, rot_ref, dest_ref):
    qk = qk_ref[0]
    rot = rot_ref[...]
    rotated = lax.dot_general(qk, rot, (((1,), (0,)), ((), ())),
                              preferred_element_type=jnp.float32,
                              precision=lax.Precision.DEFAULT)
    iota32 = lax.broadcasted_iota(jnp.int32, (S, NBK), 1)
    oh_bf, lt_f, oh_f = [], [], []
    for h in range(H):
        rh = rotated[:, h * 16:(h + 1) * 16]
        cc = jnp.concatenate([rh, -rh], axis=1)
        m = jnp.max(cc, axis=1, keepdims=True)
        bidx = jnp.min(jnp.where(cc == m, iota32, NBK), axis=1,
                       keepdims=True)
        oh = bidx == iota32
        oh_bf.append(oh.astype(jnp.bfloat16))
        oh_f.append(oh.astype(jnp.float32))
        lt_f.append((bidx < iota32).astype(jnp.float32))
    OH = jnp.concatenate(oh_bf, axis=1)
    LT = jnp.concatenate(lt_f, axis=1)
    offs = jnp.sum(LT, axis=0, keepdims=True)

    r_i = lax.broadcasted_iota(jnp.int32, (GRP, GRP), 0)
    c_i = lax.broadcasted_iota(jnp.int32, (GRP, GRP), 1)
    Ls = (r_i > c_i).astype(jnp.bfloat16)
    base = jnp.zeros((1, H * NBK), jnp.float32)
    parts = []
    for g in range(NGRP):
        blk = OH[g * GRP:(g + 1) * GRP]
        cumg = lax.dot_general(Ls, blk, (((1,), (0,)), ((), ())),
                               preferred_element_type=jnp.float32)
        parts.append(cumg + base)
        base = base + jnp.sum(blk.astype(jnp.float32), axis=0, keepdims=True)
    RANK = jnp.concatenate(parts, axis=0)
    destf = RANK + offs
    cols = []
    for h in range(H):
        sel = destf[:, h * NBK:(h + 1) * NBK] * oh_f[h]
        cols.append(jnp.sum(sel, axis=1, keepdims=True))
    dest = jnp.concatenate(cols, axis=1).astype(jnp.int32)
    dest_ref[0] = jnp.transpose(dest, (1, 0))


def _hash_dest(qk, rot):
    return pl.pallas_call(
        _hash_dest_body,
        grid=(B,),
        in_specs=[
            pl.BlockSpec((1, S, D), lambda b: (b, 0, 0)),
            pl.BlockSpec((D, H * 16), lambda b: (0, 0)),
        ],
        out_specs=pl.BlockSpec((1, H, S), lambda b: (b, 0, 0)),
        out_shape=jax.ShapeDtypeStruct((B, H, S), jnp.int32),
    )(qk, rot)


_NCHUNK = S // 128


def _make_scatter_kernel():
    mesh = plsc.VectorSubcoreMesh(core_axis_name="c", subcore_axis_name="s")

    @functools.partial(
        pl.kernel,
        mesh=mesh,
        out_type=(
            jax.ShapeDtypeStruct((B, H, S, D), jnp.float32),
            jax.ShapeDtypeStruct((B, H, S, D), jnp.float32),
            jax.ShapeDtypeStruct((B, H, S), jnp.int32),
        ),
        scratch_types=[
            pltpu.VMEM((128,), jnp.int32),
            pltpu.VMEM((128, D), jnp.float32),
            pltpu.VMEM((128, D), jnp.float32),
            pltpu.VMEM((S,), jnp.int32),
            pltpu.SemaphoreType.DMA,
            pltpu.SemaphoreType.DMA,
        ],
        compiler_params=pltpu.CompilerParams(needs_layout_passes=False),
    )
    def scatter_k(qk_hbm, v_hbm, dest_hbm, sqk_hbm, sv_hbm, st_hbm,
                  idx_v, rq_v, rv_v, st_v, sem1, sem2):
        wid = lax.axis_index("s") * 2 + lax.axis_index("c")
        for p in range(4):
            g = wid * 4 + p
            b = g // H
            h = g % H
            for c in range(_NCHUNK):
                cs = c * 128
                pltpu.sync_copy(dest_hbm.at[b, h, pl.ds(cs, 128)], idx_v)
                pltpu.sync_copy(qk_hbm.at[b, pl.ds(cs, 128)], rq_v)
                pltpu.sync_copy(v_hbm.at[b, pl.ds(cs, 128)], rv_v)
                cp1 = pltpu.async_copy(rq_v, sqk_hbm.at[b, h].at[idx_v], sem1)
                cp2 = pltpu.async_copy(rv_v, sv_hbm.at[b, h].at[idx_v], sem2)
                for k in range(8):
                    idx16 = idx_v[pl.ds(k * 16, 16)]
                    vals = (cs + k * 16) + lax.iota(jnp.int32, 16)
                    plsc.store_scatter(st_v, [idx16], vals)
                cp1.wait()
                cp2.wait()
            pltpu.sync_copy(st_v, st_hbm.at[b, h])

    return scatter_k


def _attn_body(sqk_ref, sv_ref, st_ref, hk_ref, hv_ref, ht_ref,
               so_ref, slog_ref, tx_ref):
    tx_ref[0:CH] = ht_ref[0, 0]
    tx_ref[CH:CH + S] = st_ref[0, 0]

    r_i = lax.broadcasted_iota(jnp.int32, (QB, QB + CH), 0)
    c_i = lax.broadcasted_iota(jnp.int32, (QB, QB + CH), 1)
    qchunk = (r_i // CH) * CH
    band = (c_i >= qchunk) & (c_i < qchunk + 2 * CH)
    scale = D ** -0.5

    def block(i, q, kw, vw, tq, tk):
        ssq = jnp.sum(kw * kw, axis=1, keepdims=True)
        rnorm = lax.rsqrt(jnp.maximum(ssq, 1e-24))
        kn = kw * rnorm
        dots = lax.dot_general(q * scale, kn, (((1,), (1,)), ((), ())),
                               preferred_element_type=jnp.float32)
        mask = jnp.where(band & (tq != tk), 1.0, 0.0)
        p = jnp.exp(dots) * mask
        l = jnp.sum(p, axis=1, keepdims=True)
        lse = jnp.log(l)
        o = lax.dot_general(p, vw, (((1,), (0,)), ((), ())),
                            preferred_element_type=jnp.float32)
        so_ref[0, 0, pl.ds(i * QB, QB), :] = o * (1.0 / l)
        slog_ref[0, 0, pl.ds(i * QB, QB), :] = lse

    q0 = sqk_ref[0, 0, 0:QB, :]
    kw0 = jnp.concatenate([hk_ref[0, 0], sqk_ref[0, 0, 0:QB, :]], axis=0)
    vw0 = jnp.concatenate([hv_ref[0, 0], sv_ref[0, 0, 0:QB, :]], axis=0)
    tq0 = tx_ref[CH:CH + QB]
    tk0 = jnp.transpose(tx_ref[0:QB + CH], (1, 0))
    block(0, q0, kw0, vw0, tq0, tk0)

    def step(i, _):
        q = sqk_ref[0, 0, pl.ds(i * QB, QB), :]
        kw = sqk_ref[0, 0, pl.ds(i * QB - CH, QB + CH), :]
        vw = sv_ref[0, 0, pl.ds(i * QB - CH, QB + CH), :]
        tq = tx_ref[pl.ds(CH + i * QB, QB)]
        tk = jnp.transpose(tx_ref[pl.ds(i * QB, QB + CH)], (1, 0))
        block(i, q, kw, vw, tq, tk)
        return 0

    lax.fori_loop(1, NQB, step, 0)


def _attention(sqk, sv, st2, halo_k, halo_v, halo_t2):
    return pl.pallas_call(
        _attn_body,
        grid=(B, H),
        in_specs=[
            pl.BlockSpec((1, 1, S, D), lambda b, h: (b, h, 0, 0)),
            pl.BlockSpec((1, 1, S, D), lambda b, h: (b, h, 0, 0)),
            pl.BlockSpec((1, 1, S, 1), lambda b, h: (b, h, 0, 0)),
            pl.BlockSpec((1, 1, CH, D), lambda b, h: (b, h, 0, 0)),
            pl.BlockSpec((1, 1, CH, D), lambda b, h: (b, h, 0, 0)),
            pl.BlockSpec((1, 1, CH, 1), lambda b, h: (b, h, 0, 0)),
        ],
        out_specs=[
            pl.BlockSpec((1, 1, S, D), lambda b, h: (b, h, 0, 0)),
            pl.BlockSpec((1, 1, S, 1), lambda b, h: (b, h, 0, 0)),
        ],
        out_shape=[
            jax.ShapeDtypeStruct((B, H, S, D), jnp.float32),
            jax.ShapeDtypeStruct((B, H, S, 1), jnp.float32),
        ],
        scratch_shapes=[
            pltpu.VMEM((S + CH, 1), jnp.int32),
        ],
    )(sqk, sv, st2, halo_k, halo_v, halo_t2)


def _make_gather_kernel():
    mesh = plsc.VectorSubcoreMesh(core_axis_name="c", subcore_axis_name="s")

    @functools.partial(
        pl.kernel,
        mesh=mesh,
        out_type=(
            jax.ShapeDtypeStruct((B, H, S, D), jnp.float32),
            jax.ShapeDtypeStruct((B, H, S), jnp.float32),
        ),
        scratch_types=[
            pltpu.VMEM((128,), jnp.int32),
            pltpu.VMEM((128, D), jnp.float32),
            pltpu.VMEM((S,), jnp.float32),
            pltpu.VMEM((128,), jnp.float32),
            pltpu.SemaphoreType.DMA,
        ],
        compiler_params=pltpu.CompilerParams(needs_layout_passes=False),
    )
    def gather_k(so_hbm, slog_hbm, dest_hbm, ot_hbm, lt_hbm,
                 idx_v, rows_v, sl_v, lg_v, sem):
        wid = lax.axis_index("s") * 2 + lax.axis_index("c")
        for p in range(4):
            g = wid * 4 + p
            b = g // H
            h = g % H
            pltpu.sync_copy(slog_hbm.at[b, h], sl_v)
            for c in range(_NCHUNK):
                cs = c * 128
                pltpu.sync_copy(dest_hbm.at[b, h, pl.ds(cs, 128)], idx_v)
                cp = pltpu.async_copy(so_hbm.at[b, h].at[idx_v], rows_v, sem)
                for k in range(8):
                    idx16 = idx_v[pl.ds(k * 16, 16)]
                    lg_v[pl.ds(k * 16, 16)] = plsc.load_gather(sl_v, [idx16])
                pltpu.sync_copy(lg_v, lt_hbm.at[b, h, pl.ds(cs, 128)])
                cp.wait()
                pltpu.sync_copy(rows_v, ot_hbm.at[b, h, pl.ds(cs, 128)])

    return gather_k


_TS = 512


def _combine_body(ot_ref, lt_ref, out_ref):
    lg = lt_ref[0]
    m = jnp.max(lg, axis=0, keepdims=True)
    p = jnp.exp(lg - m)
    ssum = jnp.sum(p, axis=0, keepdims=True)
    w = p / ssum
    wt = jnp.transpose(w, (1, 0))
    acc = jnp.zeros((_TS, D), jnp.float32)
    for h in range(H):
        acc = acc + ot_ref[0, h] * wt[:, h:h + 1]
    out_ref[0] = acc


def _combine(o_tok, log_tok):
    return pl.pallas_call(
        _combine_body,
        grid=(B, S // _TS),
        in_specs=[
            pl.BlockSpec((1, H, _TS, D), lambda b, t: (b, 0, t, 0)),
            pl.BlockSpec((1, H, _TS), lambda b, t: (b, 0, t)),
        ],
        out_specs=pl.BlockSpec((1, _TS, D), lambda b, t: (b, t, 0)),
        out_shape=jax.ShapeDtypeStruct((B, S, D), jnp.float32),
    )(o_tok, log_tok)


_make_scatter_kernel = functools.cache(_make_scatter_kernel)
_make_gather_kernel = functools.cache(_make_gather_kernel)


@jax.jit
def kernel(qk, v, rotations):
    rot = rotations.reshape(D, H * 16)
    dest = _hash_dest(qk, rot)
    sqk, sv, st = _make_scatter_kernel()(qk, v, dest)
    halo_k = jnp.roll(sqk[:, :, S - CH:, :], 1, axis=1)
    halo_v = jnp.roll(sv[:, :, S - CH:, :], 1, axis=1)
    halo_t = jnp.roll(st[:, :, S - CH:], 1, axis=1)
    so, slog = _attention(sqk, sv, st.reshape(B, H, S, 1),
                          halo_k, halo_v, halo_t.reshape(B, H, CH, 1))
    o_tok, log_tok = _make_gather_kernel()(so, slog.reshape(B, H, S), dest)
    return _combine(o_tok, log_tok)


# speedup vs baseline: 7.1520x; 1.1345x over previous
import functools
import jax
import jax.numpy as jnp
from jax import lax
from jax.experimental import pallas as pl
from jax.experimental.pallas import tpu as pltpu
from jax.experimental.pallas import tpu_sc as plsc

B, S, D = 16, 2048, 128
H = 8
NBK = 32
CH = 64
QB = 256
NQB = S // QB
GRP = 128
NGRP = S // GRP
SELF_VAL = -50000.0

_HIGH = lax.Precision.HIGHEST


def _hash_dest_body(qk_ref, rot_ref, dest_ref):
    qk = qk_ref[0]
    rot = rot_ref[...]
    rotated = lax.dot_general(qk, rot, (((1,), (0,)), ((), ())),
                              preferred_element_type=jnp.float32,
                              precision=lax.Precision.DEFAULT)
    iota32 = lax.broadcasted_iota(jnp.int32, (S, NBK), 1)
    oh_bf, lt_f, oh_f = [], [], []
    for h in range(H):
        rh = rotated[:, h * 16:(h + 1) * 16]
        cc = jnp.concatenate([rh, -rh], axis=1)
        m = jnp.max(cc, axis=1, keepdims=True)
        bidx = jnp.min(jnp.where(cc == m, iota32, NBK), axis=1,
                       keepdims=True)
        oh = bidx == iota32
        oh_bf.append(oh.astype(jnp.bfloat16))
        oh_f.append(oh.astype(jnp.float32))
        lt_f.append((bidx < iota32).astype(jnp.float32))
    OH = jnp.concatenate(oh_bf, axis=1)
    LT = jnp.concatenate(lt_f, axis=1)
    offs = jnp.sum(LT, axis=0, keepdims=True)

    r_i = lax.broadcasted_iota(jnp.int32, (GRP, GRP), 0)
    c_i = lax.broadcasted_iota(jnp.int32, (GRP, GRP), 1)
    Ls = (r_i > c_i).astype(jnp.bfloat16)
    base = jnp.zeros((1, H * NBK), jnp.float32)
    parts = []
    for g in range(NGRP):
        blk = OH[g * GRP:(g + 1) * GRP]
        cumg = lax.dot_general(Ls, blk, (((1,), (0,)), ((), ())),
                               preferred_element_type=jnp.float32)
        parts.append(cumg + base)
        base = base + jnp.sum(blk.astype(jnp.float32), axis=0, keepdims=True)
    RANK = jnp.concatenate(parts, axis=0)
    destf = RANK + offs
    cols = []
    for h in range(H):
        sel = destf[:, h * NBK:(h + 1) * NBK] * oh_f[h]
        cols.append(jnp.sum(sel, axis=1, keepdims=True))
    dest = jnp.concatenate(cols, axis=1).astype(jnp.int32)
    dest_ref[0] = jnp.transpose(dest, (1, 0))


def _hash_dest(qk, rot):
    return pl.pallas_call(
        _hash_dest_body,
        grid=(B,),
        in_specs=[
            pl.BlockSpec((1, S, D), lambda b: (b, 0, 0)),
            pl.BlockSpec((D, H * 16), lambda b: (0, 0)),
        ],
        out_specs=pl.BlockSpec((1, H, S), lambda b: (b, 0, 0)),
        out_shape=jax.ShapeDtypeStruct((B, H, S), jnp.int32),
    )(qk, rot)


_NCHUNK = S // 128


def _make_scatter_kernel():
    mesh = plsc.VectorSubcoreMesh(core_axis_name="c", subcore_axis_name="s")

    @functools.partial(
        pl.kernel,
        mesh=mesh,
        out_type=(
            jax.ShapeDtypeStruct((B, H, S, D), jnp.float32),
            jax.ShapeDtypeStruct((B, H, S, D), jnp.float32),
            jax.ShapeDtypeStruct((B, H, S), jnp.int32),
        ),
        scratch_types=[
            pltpu.VMEM((8, 128), jnp.int32),
            pltpu.VMEM((2, 128, D), jnp.float32),
            pltpu.VMEM((2, 128, D), jnp.float32),
            pltpu.VMEM((S,), jnp.int32),
            pltpu.VMEM((S,), jnp.int32),
            pltpu.VMEM((S,), jnp.int32),
            pltpu.VMEM((S,), jnp.int32),
            pltpu.SemaphoreType.DMA,
            pltpu.SemaphoreType.DMA,
        ],
        compiler_params=pltpu.CompilerParams(needs_layout_passes=False),
    )
    def scatter_k(qk_hbm, v_hbm, dest_hbm, sqk_hbm, sv_hbm, st_hbm,
                  idx_v, rq_v, rv_v, st0, st1, st2, st3, semL, semS):
        wid = lax.axis_index("s") * 2 + lax.axis_index("c")
        b = wid // 2
        hg = (wid % 2) * 4

        def loads(c, par):
            cs = c * 128
            hs = [pltpu.async_copy(dest_hbm.at[b, hg + j, pl.ds(cs, 128)],
                                   idx_v.at[par * 4 + j], semL) for j in range(4)]
            hs.append(pltpu.async_copy(qk_hbm.at[b, pl.ds(cs, 128)],
                                       rq_v.at[par], semL))
            hs.append(pltpu.async_copy(v_hbm.at[b, pl.ds(cs, 128)],
                                       rv_v.at[par], semL))
            return hs

        hsL = loads(0, 0)
        prevS = []
        for c in range(_NCHUNK):
            par = c % 2
            cs = c * 128
            for hh in hsL:
                hh.wait()
            for hh in prevS:
                hh.wait()
            prevS = []
            for j in range(4):
                prevS.append(pltpu.async_copy(
                    rq_v.at[par], sqk_hbm.at[b, hg + j].at[idx_v.at[par * 4 + j]],
                    semS))
                prevS.append(pltpu.async_copy(
                    rv_v.at[par], sv_hbm.at[b, hg + j].at[idx_v.at[par * 4 + j]],
                    semS))
            if c + 1 < _NCHUNK:
                hsL = loads(c + 1, 1 - par)
            for j, stj in enumerate((st0, st1, st2, st3)):
                for k in range(8):
                    idx16 = idx_v[par * 4 + j, pl.ds(k * 16, 16)]
                    vals = (cs + k * 16) + lax.iota(jnp.int32, 16)
                    plsc.store_scatter(stj, [idx16], vals)
        for hh in prevS:
            hh.wait()
        for j, stj in enumerate((st0, st1, st2, st3)):
            pltpu.sync_copy(stj, st_hbm.at[b, hg + j])

    return scatter_k


def _attn_body(sqk_ref, sv_ref, st_ref, hk_ref, hv_ref, ht_ref,
               so_ref, slog_ref, tx_ref):
    tx_ref[0:CH] = ht_ref[0, 0]
    tx_ref[CH:CH + S] = st_ref[0, 0]

    r_i = lax.broadcasted_iota(jnp.int32, (QB, QB + CH), 0)
    c_i = lax.broadcasted_iota(jnp.int32, (QB, QB + CH), 1)
    qchunk = (r_i // CH) * CH
    band = (c_i >= qchunk) & (c_i < qchunk + 2 * CH)
    scale = D ** -0.5

    def block(i, q, kw, vw, tq, tk):
        ssq = jnp.sum(kw * kw, axis=1, keepdims=True)
        rnorm = lax.rsqrt(jnp.maximum(ssq, 1e-24))
        kn = kw * rnorm
        dots = lax.dot_general(q * scale, kn, (((1,), (1,)), ((), ())),
                               preferred_element_type=jnp.float32)
        mask = jnp.where(band & (tq != tk), 1.0, 0.0)
        p = jnp.exp(dots) * mask
        l = jnp.sum(p, axis=1, keepdims=True)
        lse = jnp.log(l)
        o = lax.dot_general(p, vw, (((1,), (0,)), ((), ())),
                            preferred_element_type=jnp.float32)
        so_ref[0, 0, pl.ds(i * QB, QB), :] = o * (1.0 / l)
        slog_ref[0, 0, pl.ds(i * QB, QB), :] = lse

    q0 = sqk_ref[0, 0, 0:QB, :]
    kw0 = jnp.concatenate([hk_ref[0, 0], sqk_ref[0, 0, 0:QB, :]], axis=0)
    vw0 = jnp.concatenate([hv_ref[0, 0], sv_ref[0, 0, 0:QB, :]], axis=0)
    tq0 = tx_ref[CH:CH + QB]
    tk0 = jnp.transpose(tx_ref[0:QB + CH], (1, 0))
    block(0, q0, kw0, vw0, tq0, tk0)

    def step(i, _):
        q = sqk_ref[0, 0, pl.ds(i * QB, QB), :]
        kw = sqk_ref[0, 0, pl.ds(i * QB - CH, QB + CH), :]
        vw = sv_ref[0, 0, pl.ds(i * QB - CH, QB + CH), :]
        tq = tx_ref[pl.ds(CH + i * QB, QB)]
        tk = jnp.transpose(tx_ref[pl.ds(i * QB, QB + CH)], (1, 0))
        block(i, q, kw, vw, tq, tk)
        return 0

    lax.fori_loop(1, NQB, step, 0)


def _attention(sqk, sv, st2, halo_k, halo_v, halo_t2):
    return pl.pallas_call(
        _attn_body,
        grid=(B, H),
        in_specs=[
            pl.BlockSpec((1, 1, S, D), lambda b, h: (b, h, 0, 0)),
            pl.BlockSpec((1, 1, S, D), lambda b, h: (b, h, 0, 0)),
            pl.BlockSpec((1, 1, S, 1), lambda b, h: (b, h, 0, 0)),
            pl.BlockSpec((1, 1, CH, D), lambda b, h: (b, h, 0, 0)),
            pl.BlockSpec((1, 1, CH, D), lambda b, h: (b, h, 0, 0)),
            pl.BlockSpec((1, 1, CH, 1), lambda b, h: (b, h, 0, 0)),
        ],
        out_specs=[
            pl.BlockSpec((1, 1, S, D), lambda b, h: (b, h, 0, 0)),
            pl.BlockSpec((1, 1, S, 1), lambda b, h: (b, h, 0, 0)),
        ],
        out_shape=[
            jax.ShapeDtypeStruct((B, H, S, D), jnp.float32),
            jax.ShapeDtypeStruct((B, H, S, 1), jnp.float32),
        ],
        scratch_shapes=[
            pltpu.VMEM((S + CH, 1), jnp.int32),
        ],
    )(sqk, sv, st2, halo_k, halo_v, halo_t2)


def _make_gather_kernel():
    mesh = plsc.VectorSubcoreMesh(core_axis_name="c", subcore_axis_name="s")

    @functools.partial(
        pl.kernel,
        mesh=mesh,
        out_type=(
            jax.ShapeDtypeStruct((B, H, S, D), jnp.float32),
            jax.ShapeDtypeStruct((B, H, S), jnp.float32),
        ),
        scratch_types=[
            pltpu.VMEM((128,), jnp.int32),
            pltpu.VMEM((128, D), jnp.float32),
            pltpu.VMEM((S,), jnp.float32),
            pltpu.VMEM((128,), jnp.float32),
            pltpu.SemaphoreType.DMA,
        ],
        compiler_params=pltpu.CompilerParams(needs_layout_passes=False),
    )
    def gather_k(so_hbm, slog_hbm, dest_hbm, ot_hbm, lt_hbm,
                 idx_v, rows_v, sl_v, lg_v, sem):
        wid = lax.axis_index("s") * 2 + lax.axis_index("c")
        for p in range(4):
            g = wid * 4 + p
            b = g // H
            h = g % H
            pltpu.sync_copy(slog_hbm.at[b, h], sl_v)
            for c in range(_NCHUNK):
                cs = c * 128
                pltpu.sync_copy(dest_hbm.at[b, h, pl.ds(cs, 128)], idx_v)
                cp = pltpu.async_copy(so_hbm.at[b, h].at[idx_v], rows_v, sem)
                for k in range(8):
                    idx16 = idx_v[pl.ds(k * 16, 16)]
                    lg_v[pl.ds(k * 16, 16)] = plsc.load_gather(sl_v, [idx16])
                pltpu.sync_copy(lg_v, lt_hbm.at[b, h, pl.ds(cs, 128)])
                cp.wait()
                pltpu.sync_copy(rows_v, ot_hbm.at[b, h, pl.ds(cs, 128)])

    return gather_k


_TS = 512


def _combine_body(ot_ref, lt_ref, out_ref):
    lg = lt_ref[0]
    m = jnp.max(lg, axis=0, keepdims=True)
    p = jnp.exp(lg - m)
    ssum = jnp.sum(p, axis=0, keepdims=True)
    w = p / ssum
    wt = jnp.transpose(w, (1, 0))
    acc = jnp.zeros((_TS, D), jnp.float32)
    for h in range(H):
        acc = acc + ot_ref[0, h] * wt[:, h:h + 1]
    out_ref[0] = acc


def _combine(o_tok, log_tok):
    return pl.pallas_call(
        _combine_body,
        grid=(B, S // _TS),
        in_specs=[
            pl.BlockSpec((1, H, _TS, D), lambda b, t: (b, 0, t, 0)),
            pl.BlockSpec((1, H, _TS), lambda b, t: (b, 0, t)),
        ],
        out_specs=pl.BlockSpec((1, _TS, D), lambda b, t: (b, t, 0)),
        out_shape=jax.ShapeDtypeStruct((B, S, D), jnp.float32),
    )(o_tok, log_tok)


_make_scatter_kernel = functools.cache(_make_scatter_kernel)
_make_gather_kernel = functools.cache(_make_gather_kernel)


@jax.jit
def kernel(qk, v, rotations):
    rot = rotations.reshape(D, H * 16)
    dest = _hash_dest(qk, rot)
    sqk, sv, st = _make_scatter_kernel()(qk, v, dest)
    halo_k = jnp.roll(sqk[:, :, S - CH:, :], 1, axis=1)
    halo_v = jnp.roll(sv[:, :, S - CH:, :], 1, axis=1)
    halo_t = jnp.roll(st[:, :, S - CH:], 1, axis=1)
    so, slog = _attention(sqk, sv, st.reshape(B, H, S, 1),
                          halo_k, halo_v, halo_t.reshape(B, H, CH, 1))
    o_tok, log_tok = _make_gather_kernel()(so, slog.reshape(B, H, S), dest)
    return _combine(o_tok, log_tok)


# speedup vs baseline: 7.5277x; 1.0525x over previous
import functools
import jax
import jax.numpy as jnp
from jax import lax
from jax.experimental import pallas as pl
from jax.experimental.pallas import tpu as pltpu
from jax.experimental.pallas import tpu_sc as plsc

B, S, D = 16, 2048, 128
H = 8
NBK = 32
CH = 64
QB = 256
NQB = S // QB
GRP = 128
NGRP = S // GRP
SELF_VAL = -50000.0

_HIGH = lax.Precision.HIGHEST


def _hash_dest_body(qk_ref, rot_ref, dest_ref):
    qk = qk_ref[0]
    rot = rot_ref[...]
    rotated = lax.dot_general(qk, rot, (((1,), (0,)), ((), ())),
                              preferred_element_type=jnp.float32,
                              precision=lax.Precision.DEFAULT)
    iota32 = lax.broadcasted_iota(jnp.int32, (S, NBK), 1)
    oh_bf, lt_f, oh_f = [], [], []
    for h in range(H):
        rh = rotated[:, h * 16:(h + 1) * 16]
        cc = jnp.concatenate([rh, -rh], axis=1)
        m = jnp.max(cc, axis=1, keepdims=True)
        bidx = jnp.min(jnp.where(cc == m, iota32, NBK), axis=1,
                       keepdims=True)
        oh = bidx == iota32
        oh_bf.append(oh.astype(jnp.bfloat16))
        oh_f.append(oh.astype(jnp.float32))
        lt_f.append((bidx < iota32).astype(jnp.float32))
    OH = jnp.concatenate(oh_bf, axis=1)
    LT = jnp.concatenate(lt_f, axis=1)
    offs = jnp.sum(LT, axis=0, keepdims=True)

    r_i = lax.broadcasted_iota(jnp.int32, (GRP, GRP), 0)
    c_i = lax.broadcasted_iota(jnp.int32, (GRP, GRP), 1)
    Ls = (r_i > c_i).astype(jnp.bfloat16)
    base = jnp.zeros((1, H * NBK), jnp.float32)
    parts = []
    for g in range(NGRP):
        blk = OH[g * GRP:(g + 1) * GRP]
        cumg = lax.dot_general(Ls, blk, (((1,), (0,)), ((), ())),
                               preferred_element_type=jnp.float32)
        parts.append(cumg + base)
        base = base + jnp.sum(blk.astype(jnp.float32), axis=0, keepdims=True)
    RANK = jnp.concatenate(parts, axis=0)
    destf = RANK + offs
    cols = []
    for h in range(H):
        sel = destf[:, h * NBK:(h + 1) * NBK] * oh_f[h]
        cols.append(jnp.sum(sel, axis=1, keepdims=True))
    dest = jnp.concatenate(cols, axis=1).astype(jnp.int32)
    dest_ref[0] = jnp.transpose(dest, (1, 0))


def _hash_dest(qk, rot):
    return pl.pallas_call(
        _hash_dest_body,
        grid=(B,),
        in_specs=[
            pl.BlockSpec((1, S, D), lambda b: (b, 0, 0)),
            pl.BlockSpec((D, H * 16), lambda b: (0, 0)),
        ],
        out_specs=pl.BlockSpec((1, H, S), lambda b: (b, 0, 0)),
        out_shape=jax.ShapeDtypeStruct((B, H, S), jnp.int32),
    )(qk, rot)


_NCHUNK = S // 128


def _make_scatter_kernel():
    mesh = plsc.VectorSubcoreMesh(core_axis_name="c", subcore_axis_name="s")

    @functools.partial(
        pl.kernel,
        mesh=mesh,
        out_type=(
            jax.ShapeDtypeStruct((B, H, S, D), jnp.float32),
            jax.ShapeDtypeStruct((B, H, S, D), jnp.float32),
            jax.ShapeDtypeStruct((B, H, S), jnp.int32),
        ),
        scratch_types=[
            pltpu.VMEM((8, 128), jnp.int32),
            pltpu.VMEM((2, 128, D), jnp.float32),
            pltpu.VMEM((2, 128, D), jnp.float32),
            pltpu.VMEM((S,), jnp.int32),
            pltpu.VMEM((S,), jnp.int32),
            pltpu.VMEM((S,), jnp.int32),
            pltpu.VMEM((S,), jnp.int32),
            pltpu.SemaphoreType.DMA,
            pltpu.SemaphoreType.DMA,
        ],
        compiler_params=pltpu.CompilerParams(needs_layout_passes=False),
    )
    def scatter_k(qk_hbm, v_hbm, dest_hbm, sqk_hbm, sv_hbm, st_hbm,
                  idx_v, rq_v, rv_v, st0, st1, st2, st3, semL, semS):
        wid = lax.axis_index("s") * 2 + lax.axis_index("c")
        b = wid // 2
        hg = (wid % 2) * 4

        def loads(c, par):
            cs = c * 128
            hs = [pltpu.async_copy(dest_hbm.at[b, hg + j, pl.ds(cs, 128)],
                                   idx_v.at[par * 4 + j], semL) for j in range(4)]
            hs.append(pltpu.async_copy(qk_hbm.at[b, pl.ds(cs, 128)],
                                       rq_v.at[par], semL))
            hs.append(pltpu.async_copy(v_hbm.at[b, pl.ds(cs, 128)],
                                       rv_v.at[par], semL))
            return hs

        hsL = loads(0, 0)
        prevS = []
        for c in range(_NCHUNK):
            par = c % 2
            cs = c * 128
            for hh in hsL:
                hh.wait()
            for hh in prevS:
                hh.wait()
            prevS = []
            for j in range(4):
                prevS.append(pltpu.async_copy(
                    rq_v.at[par], sqk_hbm.at[b, hg + j].at[idx_v.at[par * 4 + j]],
                    semS))
                prevS.append(pltpu.async_copy(
                    rv_v.at[par], sv_hbm.at[b, hg + j].at[idx_v.at[par * 4 + j]],
                    semS))
            if c + 1 < _NCHUNK:
                hsL = loads(c + 1, 1 - par)
            for j, stj in enumerate((st0, st1, st2, st3)):
                for k in range(8):
                    idx16 = idx_v[par * 4 + j, pl.ds(k * 16, 16)]
                    vals = (cs + k * 16) + lax.iota(jnp.int32, 16)
                    plsc.store_scatter(stj, [idx16], vals)
        for hh in prevS:
            hh.wait()
        for j, stj in enumerate((st0, st1, st2, st3)):
            pltpu.sync_copy(stj, st_hbm.at[b, hg + j])

    return scatter_k


def _attn_body(sqk_ref, sv_ref, st_ref, hk_ref, hv_ref, ht_ref,
               so_ref, slog_ref, tx_ref):
    tx_ref[0:CH] = ht_ref[0, 0]
    tx_ref[CH:CH + S] = st_ref[0, 0]

    r_i = lax.broadcasted_iota(jnp.int32, (QB, QB + CH), 0)
    c_i = lax.broadcasted_iota(jnp.int32, (QB, QB + CH), 1)
    qchunk = (r_i // CH) * CH
    band = (c_i >= qchunk) & (c_i < qchunk + 2 * CH)
    scale = D ** -0.5

    def block(i, q, kw, vw, tq, tk):
        ssq = jnp.sum(kw * kw, axis=1, keepdims=True)
        rnorm = lax.rsqrt(jnp.maximum(ssq, 1e-24))
        kn = kw * rnorm
        dots = lax.dot_general(q * scale, kn, (((1,), (1,)), ((), ())),
                               preferred_element_type=jnp.float32)
        mask = jnp.where(band & (tq != tk), 1.0, 0.0)
        p = jnp.exp(dots) * mask
        l = jnp.sum(p, axis=1, keepdims=True)
        lse = jnp.log(l)
        o = lax.dot_general(p, vw, (((1,), (0,)), ((), ())),
                            preferred_element_type=jnp.float32)
        so_ref[0, 0, pl.ds(i * QB, QB), :] = o * (1.0 / l)
        slog_ref[0, 0, pl.ds(i * QB, QB), :] = lse

    q0 = sqk_ref[0, 0, 0:QB, :]
    kw0 = jnp.concatenate([hk_ref[0, 0], sqk_ref[0, 0, 0:QB, :]], axis=0)
    vw0 = jnp.concatenate([hv_ref[0, 0], sv_ref[0, 0, 0:QB, :]], axis=0)
    tq0 = tx_ref[CH:CH + QB]
    tk0 = jnp.transpose(tx_ref[0:QB + CH], (1, 0))
    block(0, q0, kw0, vw0, tq0, tk0)

    def step(i, _):
        q = sqk_ref[0, 0, pl.ds(i * QB, QB), :]
        kw = sqk_ref[0, 0, pl.ds(i * QB - CH, QB + CH), :]
        vw = sv_ref[0, 0, pl.ds(i * QB - CH, QB + CH), :]
        tq = tx_ref[pl.ds(CH + i * QB, QB)]
        tk = jnp.transpose(tx_ref[pl.ds(i * QB, QB + CH)], (1, 0))
        block(i, q, kw, vw, tq, tk)
        return 0

    lax.fori_loop(1, NQB, step, 0)


def _attention(sqk, sv, st2, halo_k, halo_v, halo_t2):
    return pl.pallas_call(
        _attn_body,
        grid=(B, H),
        in_specs=[
            pl.BlockSpec((1, 1, S, D), lambda b, h: (b, h, 0, 0)),
            pl.BlockSpec((1, 1, S, D), lambda b, h: (b, h, 0, 0)),
            pl.BlockSpec((1, 1, S, 1), lambda b, h: (b, h, 0, 0)),
            pl.BlockSpec((1, 1, CH, D), lambda b, h: (b, h, 0, 0)),
            pl.BlockSpec((1, 1, CH, D), lambda b, h: (b, h, 0, 0)),
            pl.BlockSpec((1, 1, CH, 1), lambda b, h: (b, h, 0, 0)),
        ],
        out_specs=[
            pl.BlockSpec((1, 1, S, D), lambda b, h: (b, h, 0, 0)),
            pl.BlockSpec((1, 1, S, 1), lambda b, h: (b, h, 0, 0)),
        ],
        out_shape=[
            jax.ShapeDtypeStruct((B, H, S, D), jnp.float32),
            jax.ShapeDtypeStruct((B, H, S, 1), jnp.float32),
        ],
        scratch_shapes=[
            pltpu.VMEM((S + CH, 1), jnp.int32),
        ],
    )(sqk, sv, st2, halo_k, halo_v, halo_t2)


def _make_gather_kernel():
    mesh = plsc.VectorSubcoreMesh(core_axis_name="c", subcore_axis_name="s")

    @functools.partial(
        pl.kernel,
        mesh=mesh,
        out_type=(
            jax.ShapeDtypeStruct((B, H, S, D), jnp.float32),
            jax.ShapeDtypeStruct((B, H, S), jnp.float32),
        ),
        scratch_types=[
            pltpu.VMEM((2, 128), jnp.int32),
            pltpu.VMEM((2, 128, D), jnp.float32),
            pltpu.VMEM((S,), jnp.float32),
            pltpu.VMEM((2, 128), jnp.float32),
            pltpu.SemaphoreType.DMA,
            pltpu.SemaphoreType.DMA,
            pltpu.SemaphoreType.DMA,
        ],
        compiler_params=pltpu.CompilerParams(needs_layout_passes=False),
    )
    def gather_k(so_hbm, slog_hbm, dest_hbm, ot_hbm, lt_hbm,
                 idx_v, rows_v, sl_v, lg_v, semI, semG, semO):
        wid = lax.axis_index("s") * 2 + lax.axis_index("c")
        for p in range(4):
            g = wid * 4 + p
            b = g // H
            h = g % H
            pltpu.sync_copy(slog_hbm.at[b, h], sl_v)
            idxh = pltpu.async_copy(dest_hbm.at[b, h, pl.ds(0, 128)],
                                    idx_v.at[0], semI)
            gh_prev = None
            out_prev = []
            for c in range(_NCHUNK):
                par = c % 2
                idxh.wait()
                for hh in out_prev:
                    hh.wait()
                gh = pltpu.async_copy(so_hbm.at[b, h].at[idx_v.at[par]],
                                      rows_v.at[par], semG)
                if gh_prev is not None:
                    for k in range(8):
                        idx16 = idx_v[1 - par, pl.ds(k * 16, 16)]
                        lg_v[1 - par, pl.ds(k * 16, 16)] = (
                            plsc.load_gather(sl_v, [idx16]))
                    gh_prev.wait()
                    ps = (c - 1) * 128
                    out_prev = [
                        pltpu.async_copy(rows_v.at[1 - par],
                                         ot_hbm.at[b, h, pl.ds(ps, 128)],
                                         semO),
                        pltpu.async_copy(lg_v.at[1 - par],
                                         lt_hbm.at[b, h, pl.ds(ps, 128)],
                                         semO),
                    ]
                    if c + 1 < _NCHUNK:
                        idxh = pltpu.async_copy(
                            dest_hbm.at[b, h, pl.ds((c + 1) * 128, 128)],
                            idx_v.at[1 - par], semI)
                elif c + 1 < _NCHUNK:
                    idxh = pltpu.async_copy(
                        dest_hbm.at[b, h, pl.ds((c + 1) * 128, 128)],
                        idx_v.at[1 - par], semI)
                gh_prev = gh
            par = (_NCHUNK - 1) % 2
            for k in range(8):
                idx16 = idx_v[par, pl.ds(k * 16, 16)]
                lg_v[par, pl.ds(k * 16, 16)] = plsc.load_gather(sl_v, [idx16])
            gh_prev.wait()
            ps = (_NCHUNK - 1) * 128
            for hh in out_prev:
                hh.wait()
            pltpu.sync_copy(rows_v.at[par], ot_hbm.at[b, h, pl.ds(ps, 128)])
            pltpu.sync_copy(lg_v.at[par], lt_hbm.at[b, h, pl.ds(ps, 128)])

    return gather_k


_TS = 512


def _combine_body(ot_ref, lt_ref, out_ref):
    lg = lt_ref[0]
    m = jnp.max(lg, axis=0, keepdims=True)
    p = jnp.exp(lg - m)
    ssum = jnp.sum(p, axis=0, keepdims=True)
    w = p / ssum
    wt = jnp.transpose(w, (1, 0))
    acc = jnp.zeros((_TS, D), jnp.float32)
    for h in range(H):
        acc = acc + ot_ref[0, h] * wt[:, h:h + 1]
    out_ref[0] = acc


def _combine(o_tok, log_tok):
    return pl.pallas_call(
        _combine_body,
        grid=(B, S // _TS),
        in_specs=[
            pl.BlockSpec((1, H, _TS, D), lambda b, t: (b, 0, t, 0)),
            pl.BlockSpec((1, H, _TS), lambda b, t: (b, 0, t)),
        ],
        out_specs=pl.BlockSpec((1, _TS, D), lambda b, t: (b, t, 0)),
        out_shape=jax.ShapeDtypeStruct((B, S, D), jnp.float32),
    )(o_tok, log_tok)


_make_scatter_kernel = functools.cache(_make_scatter_kernel)
_make_gather_kernel = functools.cache(_make_gather_kernel)


@jax.jit
def kernel(qk, v, rotations):
    rot = rotations.reshape(D, H * 16)
    dest = _hash_dest(qk, rot)
    sqk, sv, st = _make_scatter_kernel()(qk, v, dest)
    halo_k = jnp.roll(sqk[:, :, S - CH:, :], 1, axis=1)
    halo_v = jnp.roll(sv[:, :, S - CH:, :], 1, axis=1)
    halo_t = jnp.roll(st[:, :, S - CH:], 1, axis=1)
    so, slog = _attention(sqk, sv, st.reshape(B, H, S, 1),
                          halo_k, halo_v, halo_t.reshape(B, H, CH, 1))
    o_tok, log_tok = _make_gather_kernel()(so, slog.reshape(B, H, S), dest)
    return _combine(o_tok, log_tok)


# speedup vs baseline: 8.9003x; 1.1823x over previous
import functools
import jax
import jax.numpy as jnp
from jax import lax
from jax.experimental import pallas as pl
from jax.experimental.pallas import tpu as pltpu
from jax.experimental.pallas import tpu_sc as plsc

B, S, D = 16, 2048, 128
H = 8
NBK = 32
CH = 64
QB = 256
NQB = S // QB
GRP = 128
NGRP = S // GRP
SELF_VAL = -50000.0

_HIGH = lax.Precision.HIGHEST


def _hash_dest_body(qk_ref, rot_ref, dest_ref):
    qk = qk_ref[0]
    rot = rot_ref[...]
    rotated = lax.dot_general(qk, rot, (((1,), (0,)), ((), ())),
                              preferred_element_type=jnp.float32,
                              precision=lax.Precision.DEFAULT)
    iota32 = lax.broadcasted_iota(jnp.int32, (S, NBK), 1)
    oh_bf, lt_f, oh_f = [], [], []
    for h in range(H):
        rh = rotated[:, h * 16:(h + 1) * 16]
        cc = jnp.concatenate([rh, -rh], axis=1)
        m = jnp.max(cc, axis=1, keepdims=True)
        bidx = jnp.min(jnp.where(cc == m, iota32, NBK), axis=1,
                       keepdims=True)
        oh = bidx == iota32
        oh_bf.append(oh.astype(jnp.bfloat16))
        oh_f.append(oh.astype(jnp.float32))
        lt_f.append((bidx < iota32).astype(jnp.float32))
    OH = jnp.concatenate(oh_bf, axis=1)
    LT = jnp.concatenate(lt_f, axis=1)
    offs = jnp.sum(LT, axis=0, keepdims=True)

    r_i = lax.broadcasted_iota(jnp.int32, (GRP, GRP), 0)
    c_i = lax.broadcasted_iota(jnp.int32, (GRP, GRP), 1)
    Ls = (r_i > c_i).astype(jnp.bfloat16)
    base = jnp.zeros((1, H * NBK), jnp.float32)
    parts = []
    for g in range(NGRP):
        blk = OH[g * GRP:(g + 1) * GRP]
        cumg = lax.dot_general(Ls, blk, (((1,), (0,)), ((), ())),
                               preferred_element_type=jnp.float32)
        parts.append(cumg + base)
        base = base + jnp.sum(blk.astype(jnp.float32), axis=0, keepdims=True)
    RANK = jnp.concatenate(parts, axis=0)
    destf = RANK + offs
    cols = []
    for h in range(H):
        sel = destf[:, h * NBK:(h + 1) * NBK] * oh_f[h]
        cols.append(jnp.sum(sel, axis=1, keepdims=True))
    dest = jnp.concatenate(cols, axis=1).astype(jnp.int32)
    dest_ref[0] = jnp.transpose(dest, (1, 0))


def _hash_dest(qk, rot):
    return pl.pallas_call(
        _hash_dest_body,
        grid=(B,),
        in_specs=[
            pl.BlockSpec((1, S, D), lambda b: (b, 0, 0)),
            pl.BlockSpec((D, H * 16), lambda b: (0, 0)),
        ],
        out_specs=pl.BlockSpec((1, H, S), lambda b: (b, 0, 0)),
        out_shape=jax.ShapeDtypeStruct((B, H, S), jnp.int32),
    )(qk, rot)


_NCHUNK = S // 128


def _make_scatter_kernel():
    mesh = plsc.VectorSubcoreMesh(core_axis_name="c", subcore_axis_name="s")

    @functools.partial(
        pl.kernel,
        mesh=mesh,
        out_type=(
            jax.ShapeDtypeStruct((B, H, S, D), jnp.float32),
            jax.ShapeDtypeStruct((B, H, S, D), jnp.float32),
            jax.ShapeDtypeStruct((B, H, S), jnp.int32),
        ),
        scratch_types=[
            pltpu.VMEM((8, 128), jnp.int32),
            pltpu.VMEM((2, 128, D), jnp.float32),
            pltpu.VMEM((2, 128, D), jnp.float32),
            pltpu.VMEM((S,), jnp.int32),
            pltpu.VMEM((S,), jnp.int32),
            pltpu.VMEM((S,), jnp.int32),
            pltpu.VMEM((S,), jnp.int32),
            pltpu.SemaphoreType.DMA,
            pltpu.SemaphoreType.DMA,
        ],
        compiler_params=pltpu.CompilerParams(needs_layout_passes=False),
    )
    def scatter_k(qk_hbm, v_hbm, dest_hbm, sqk_hbm, sv_hbm, st_hbm,
                  idx_v, rq_v, rv_v, st0, st1, st2, st3, semL, semS):
        wid = lax.axis_index("s") * 2 + lax.axis_index("c")
        b = wid // 2
        hg = (wid % 2) * 4

        def loads(c, par):
            cs = c * 128
            hs = [pltpu.async_copy(dest_hbm.at[b, hg + j, pl.ds(cs, 128)],
                                   idx_v.at[par * 4 + j], semL) for j in range(4)]
            hs.append(pltpu.async_copy(qk_hbm.at[b, pl.ds(cs, 128)],
                                       rq_v.at[par], semL))
            hs.append(pltpu.async_copy(v_hbm.at[b, pl.ds(cs, 128)],
                                       rv_v.at[par], semL))
            return hs

        hsL = loads(0, 0)
        prevS = []
        for c in range(_NCHUNK):
            par = c % 2
            cs = c * 128
            for hh in hsL:
                hh.wait()
            for hh in prevS:
                hh.wait()
            prevS = []
            for j in range(4):
                prevS.append(pltpu.async_copy(
                    rq_v.at[par], sqk_hbm.at[b, hg + j].at[idx_v.at[par * 4 + j]],
                    semS))
                prevS.append(pltpu.async_copy(
                    rv_v.at[par], sv_hbm.at[b, hg + j].at[idx_v.at[par * 4 + j]],
                    semS))
            if c + 1 < _NCHUNK:
                hsL = loads(c + 1, 1 - par)
            for j, stj in enumerate((st0, st1, st2, st3)):
                for k in range(8):
                    idx16 = idx_v[par * 4 + j, pl.ds(k * 16, 16)]
                    vals = (cs + k * 16) + lax.iota(jnp.int32, 16)
                    plsc.store_scatter(stj, [idx16], vals)
        for hh in prevS:
            hh.wait()
        for j, stj in enumerate((st0, st1, st2, st3)):
            pltpu.sync_copy(stj, st_hbm.at[b, hg + j])

    return scatter_k


def _attn_body(sqk_ref, sv_ref, st_ref, hk_ref, hv_ref, ht_ref,
               so_ref, slog_ref, tx_ref):
    tx_ref[0:CH] = ht_ref[0, 0]
    tx_ref[CH:CH + S] = st_ref[0, 0]

    r_i = lax.broadcasted_iota(jnp.int32, (QB, QB + CH), 0)
    c_i = lax.broadcasted_iota(jnp.int32, (QB, QB + CH), 1)
    qchunk = (r_i // CH) * CH
    band = (c_i >= qchunk) & (c_i < qchunk + 2 * CH)
    scale = D ** -0.5

    def block(i, q, kw, vw, tq, tk):
        ssq = jnp.sum(kw * kw, axis=1, keepdims=True)
        rnorm = lax.rsqrt(jnp.maximum(ssq, 1e-24))
        kn = kw * rnorm
        dots = lax.dot_general(q * scale, kn, (((1,), (1,)), ((), ())),
                               preferred_element_type=jnp.float32)
        mask = jnp.where(band & (tq != tk), 1.0, 0.0)
        p = jnp.exp(dots) * mask
        l = jnp.sum(p, axis=1, keepdims=True)
        lse = jnp.log(l)
        o = lax.dot_general(p, vw, (((1,), (0,)), ((), ())),
                            preferred_element_type=jnp.float32)
        so_ref[0, 0, pl.ds(i * QB, QB), :] = o * (1.0 / l)
        slog_ref[0, 0, pl.ds(i * QB, QB), :] = lse

    q0 = sqk_ref[0, 0, 0:QB, :]
    kw0 = jnp.concatenate([hk_ref[0, 0], sqk_ref[0, 0, 0:QB, :]], axis=0)
    vw0 = jnp.concatenate([hv_ref[0, 0], sv_ref[0, 0, 0:QB, :]], axis=0)
    tq0 = tx_ref[CH:CH + QB]
    tk0 = jnp.transpose(tx_ref[0:QB + CH], (1, 0))
    block(0, q0, kw0, vw0, tq0, tk0)

    for i in range(1, NQB):
        q = sqk_ref[0, 0, pl.ds(i * QB, QB), :]
        kw = sqk_ref[0, 0, pl.ds(i * QB - CH, QB + CH), :]
        vw = sv_ref[0, 0, pl.ds(i * QB - CH, QB + CH), :]
        tq = tx_ref[pl.ds(CH + i * QB, QB)]
        tk = jnp.transpose(tx_ref[pl.ds(i * QB, QB + CH)], (1, 0))
        block(i, q, kw, vw, tq, tk)


def _attention(sqk, sv, st2, halo_k, halo_v, halo_t2):
    return pl.pallas_call(
        _attn_body,
        grid=(B, H),
        in_specs=[
            pl.BlockSpec((1, 1, S, D), lambda b, h: (b, h, 0, 0)),
            pl.BlockSpec((1, 1, S, D), lambda b, h: (b, h, 0, 0)),
            pl.BlockSpec((1, 1, S, 1), lambda b, h: (b, h, 0, 0)),
            pl.BlockSpec((1, 1, CH, D), lambda b, h: (b, h, 0, 0)),
            pl.BlockSpec((1, 1, CH, D), lambda b, h: (b, h, 0, 0)),
            pl.BlockSpec((1, 1, CH, 1), lambda b, h: (b, h, 0, 0)),
        ],
        out_specs=[
            pl.BlockSpec((1, 1, S, D), lambda b, h: (b, h, 0, 0)),
            pl.BlockSpec((1, 1, S, 1), lambda b, h: (b, h, 0, 0)),
        ],
        out_shape=[
            jax.ShapeDtypeStruct((B, H, S, D), jnp.float32),
            jax.ShapeDtypeStruct((B, H, S, 1), jnp.float32),
        ],
        scratch_shapes=[
            pltpu.VMEM((S + CH, 1), jnp.int32),
        ],
    )(sqk, sv, st2, halo_k, halo_v, halo_t2)


def _make_gather_kernel():
    mesh = plsc.VectorSubcoreMesh(core_axis_name="c", subcore_axis_name="s")

    @functools.partial(
        pl.kernel,
        mesh=mesh,
        out_type=(
            jax.ShapeDtypeStruct((B, H, S, D), jnp.float32),
            jax.ShapeDtypeStruct((B, H, S), jnp.float32),
        ),
        scratch_types=[
            pltpu.VMEM((2, 128), jnp.int32),
            pltpu.VMEM((2, 128, D), jnp.float32),
            pltpu.VMEM((S,), jnp.float32),
            pltpu.VMEM((2, 128), jnp.float32),
            pltpu.SemaphoreType.DMA,
            pltpu.SemaphoreType.DMA,
            pltpu.SemaphoreType.DMA,
        ],
        compiler_params=pltpu.CompilerParams(needs_layout_passes=False),
    )
    def gather_k(so_hbm, slog_hbm, dest_hbm, ot_hbm, lt_hbm,
                 idx_v, rows_v, sl_v, lg_v, semI, semG, semO):
        wid = lax.axis_index("s") * 2 + lax.axis_index("c")
        for p in range(4):
            g = wid * 4 + p
            b = g // H
            h = g % H
            pltpu.sync_copy(slog_hbm.at[b, h], sl_v)
            idxh = pltpu.async_copy(dest_hbm.at[b, h, pl.ds(0, 128)],
                                    idx_v.at[0], semI)
            gh_prev = None
            out_prev = []
            for c in range(_NCHUNK):
                par = c % 2
                idxh.wait()
                for hh in out_prev:
                    hh.wait()
                gh = pltpu.async_copy(so_hbm.at[b, h].at[idx_v.at[par]],
                                      rows_v.at[par], semG)
                if gh_prev is not None:
                    for k in range(8):
                        idx16 = idx_v[1 - par, pl.ds(k * 16, 16)]
                        lg_v[1 - par, pl.ds(k * 16, 16)] = (
                            plsc.load_gather(sl_v, [idx16]))
                    gh_prev.wait()
                    ps = (c - 1) * 128
                    out_prev = [
                        pltpu.async_copy(rows_v.at[1 - par],
                                         ot_hbm.at[b, h, pl.ds(ps, 128)],
                                         semO),
                        pltpu.async_copy(lg_v.at[1 - par],
                                         lt_hbm.at[b, h, pl.ds(ps, 128)],
                                         semO),
                    ]
                    if c + 1 < _NCHUNK:
                        idxh = pltpu.async_copy(
                            dest_hbm.at[b, h, pl.ds((c + 1) * 128, 128)],
                            idx_v.at[1 - par], semI)
                elif c + 1 < _NCHUNK:
                    idxh = pltpu.async_copy(
                        dest_hbm.at[b, h, pl.ds((c + 1) * 128, 128)],
                        idx_v.at[1 - par], semI)
                gh_prev = gh
            par = (_NCHUNK - 1) % 2
            for k in range(8):
                idx16 = idx_v[par, pl.ds(k * 16, 16)]
                lg_v[par, pl.ds(k * 16, 16)] = plsc.load_gather(sl_v, [idx16])
            gh_prev.wait()
            ps = (_NCHUNK - 1) * 128
            for hh in out_prev:
                hh.wait()
            pltpu.sync_copy(rows_v.at[par], ot_hbm.at[b, h, pl.ds(ps, 128)])
            pltpu.sync_copy(lg_v.at[par], lt_hbm.at[b, h, pl.ds(ps, 128)])

    return gather_k


_TS = 512


def _combine_body(ot_ref, lt_ref, out_ref):
    lg = lt_ref[0]
    m = jnp.max(lg, axis=0, keepdims=True)
    p = jnp.exp(lg - m)
    ssum = jnp.sum(p, axis=0, keepdims=True)
    w = p / ssum
    wt = jnp.transpose(w, (1, 0))
    acc = jnp.zeros((_TS, D), jnp.float32)
    for h in range(H):
        acc = acc + ot_ref[0, h] * wt[:, h:h + 1]
    out_ref[0] = acc


def _combine(o_tok, log_tok):
    return pl.pallas_call(
        _combine_body,
        grid=(B, S // _TS),
        in_specs=[
            pl.BlockSpec((1, H, _TS, D), lambda b, t: (b, 0, t, 0)),
            pl.BlockSpec((1, H, _TS), lambda b, t: (b, 0, t)),
        ],
        out_specs=pl.BlockSpec((1, _TS, D), lambda b, t: (b, t, 0)),
        out_shape=jax.ShapeDtypeStruct((B, S, D), jnp.float32),
    )(o_tok, log_tok)


_make_scatter_kernel = functools.cache(_make_scatter_kernel)
_make_gather_kernel = functools.cache(_make_gather_kernel)


@jax.jit
def kernel(qk, v, rotations):
    rot = rotations.reshape(D, H * 16)
    dest = _hash_dest(qk, rot)
    sqk, sv, st = _make_scatter_kernel()(qk, v, dest)
    halo_k = jnp.roll(sqk[:, :, S - CH:, :], 1, axis=1)
    halo_v = jnp.roll(sv[:, :, S - CH:, :], 1, axis=1)
    halo_t = jnp.roll(st[:, :, S - CH:], 1, axis=1)
    so, slog = _attention(sqk, sv, st.reshape(B, H, S, 1),
                          halo_k, halo_v, halo_t.reshape(B, H, CH, 1))
    o_tok, log_tok = _make_gather_kernel()(so, slog.reshape(B, H, S), dest)
    return _combine(o_tok, log_tok)


# speedup vs baseline: 9.6930x; 1.0891x over previous
import functools
import jax
import jax.numpy as jnp
from jax import lax
from jax.experimental import pallas as pl
from jax.experimental.pallas import tpu as pltpu
from jax.experimental.pallas import tpu_sc as plsc

B, S, D = 16, 2048, 128
H = 8
NBK = 32
CH = 64
QB = 256
NQB = S // QB
GRP = 128
NGRP = S // GRP
SELF_VAL = -50000.0

_HIGH = lax.Precision.HIGHEST


def _hash_dest_body(qk_ref, rot_ref, dest_ref):
    qk = qk_ref[0]
    rot = rot_ref[...]
    rotated = lax.dot_general(qk, rot, (((1,), (0,)), ((), ())),
                              preferred_element_type=jnp.float32,
                              precision=lax.Precision.DEFAULT)
    ohm_bf = []
    for h in range(H):
        rh = rotated[:, h * 16:(h + 1) * 16]
        cc = jnp.concatenate([rh, -rh], axis=1)
        m = jnp.max(cc, axis=1, keepdims=True)
        ohm_bf.append((cc == m).astype(jnp.bfloat16))
    OHM = jnp.concatenate(ohm_bf, axis=1)
    a_i = lax.broadcasted_iota(jnp.int32, (H * NBK, H * NBK), 0)
    b_i = lax.broadcasted_iota(jnp.int32, (H * NBK, H * NBK), 1)
    U = ((a_i // NBK == b_i // NBK) & (a_i < b_i)).astype(jnp.bfloat16)
    prem = lax.dot_general(OHM, U, (((1,), (0,)), ((), ())),
                           preferred_element_type=jnp.float32)
    oh_f256 = jnp.where(prem == 0.0, OHM.astype(jnp.float32), 0.0)
    OH = oh_f256.astype(jnp.bfloat16)
    LT = lax.dot_general(OH, U, (((1,), (0,)), ((), ())),
                         preferred_element_type=jnp.float32)
    offs = jnp.sum(LT, axis=0, keepdims=True)
    oh_f = [oh_f256[:, h * NBK:(h + 1) * NBK] for h in range(H)]

    r_i = lax.broadcasted_iota(jnp.int32, (GRP, GRP), 0)
    c_i = lax.broadcasted_iota(jnp.int32, (GRP, GRP), 1)
    Ls = (r_i > c_i).astype(jnp.bfloat16)
    base = jnp.zeros((1, H * NBK), jnp.float32)
    parts = []
    for g in range(NGRP):
        blk = OH[g * GRP:(g + 1) * GRP]
        cumg = lax.dot_general(Ls, blk, (((1,), (0,)), ((), ())),
                               preferred_element_type=jnp.float32)
        parts.append(cumg + base)
        base = base + jnp.sum(blk.astype(jnp.float32), axis=0, keepdims=True)
    RANK = jnp.concatenate(parts, axis=0)
    destf = RANK + offs
    cols = []
    for h in range(H):
        sel = destf[:, h * NBK:(h + 1) * NBK] * oh_f[h]
        cols.append(jnp.sum(sel, axis=1, keepdims=True))
    dest = jnp.concatenate(cols, axis=1).astype(jnp.int32)
    dest_ref[0] = jnp.transpose(dest, (1, 0))


def _hash_dest(qk, rot):
    return pl.pallas_call(
        _hash_dest_body,
        grid=(B,),
        in_specs=[
            pl.BlockSpec((1, S, D), lambda b: (b, 0, 0)),
            pl.BlockSpec((D, H * 16), lambda b: (0, 0)),
        ],
        out_specs=pl.BlockSpec((1, H, S), lambda b: (b, 0, 0)),
        out_shape=jax.ShapeDtypeStruct((B, H, S), jnp.int32),
    )(qk, rot)


_NCHUNK = S // 128


def _make_scatter_kernel():
    mesh = plsc.VectorSubcoreMesh(core_axis_name="c", subcore_axis_name="s")

    @functools.partial(
        pl.kernel,
        mesh=mesh,
        out_type=(
            jax.ShapeDtypeStruct((B, H, S, D), jnp.float32),
            jax.ShapeDtypeStruct((B, H, S, D), jnp.float32),
            jax.ShapeDtypeStruct((B, H, S), jnp.int32),
        ),
        scratch_types=[
            pltpu.VMEM((8, 128), jnp.int32),
            pltpu.VMEM((2, 128, D), jnp.float32),
            pltpu.VMEM((2, 128, D), jnp.float32),
            pltpu.VMEM((S,), jnp.int32),
            pltpu.VMEM((S,), jnp.int32),
            pltpu.VMEM((S,), jnp.int32),
            pltpu.VMEM((S,), jnp.int32),
            pltpu.SemaphoreType.DMA,
            pltpu.SemaphoreType.DMA,
        ],
        compiler_params=pltpu.CompilerParams(needs_layout_passes=False),
    )
    def scatter_k(qk_hbm, v_hbm, dest_hbm, sqk_hbm, sv_hbm, st_hbm,
                  idx_v, rq_v, rv_v, st0, st1, st2, st3, semL, semS):
        wid = lax.axis_index("s") * 2 + lax.axis_index("c")
        b = wid // 2
        hg = (wid % 2) * 4

        def loads(c, par):
            cs = c * 128
            hs = [pltpu.async_copy(dest_hbm.at[b, hg + j, pl.ds(cs, 128)],
                                   idx_v.at[par * 4 + j], semL) for j in range(4)]
            hs.append(pltpu.async_copy(qk_hbm.at[b, pl.ds(cs, 128)],
                                       rq_v.at[par], semL))
            hs.append(pltpu.async_copy(v_hbm.at[b, pl.ds(cs, 128)],
                                       rv_v.at[par], semL))
            return hs

        hsL = loads(0, 0)
        prevS = []
        for c in range(_NCHUNK):
            par = c % 2
            cs = c * 128
            for hh in hsL:
                hh.wait()
            for hh in prevS:
                hh.wait()
            prevS = []
            for j in range(4):
                prevS.append(pltpu.async_copy(
                    rq_v.at[par], sqk_hbm.at[b, hg + j].at[idx_v.at[par * 4 + j]],
                    semS))
                prevS.append(pltpu.async_copy(
                    rv_v.at[par], sv_hbm.at[b, hg + j].at[idx_v.at[par * 4 + j]],
                    semS))
            if c + 1 < _NCHUNK:
                hsL = loads(c + 1, 1 - par)
            for j, stj in enumerate((st0, st1, st2, st3)):
                for k in range(8):
                    idx16 = idx_v[par * 4 + j, pl.ds(k * 16, 16)]
                    vals = (cs + k * 16) + lax.iota(jnp.int32, 16)
                    plsc.store_scatter(stj, [idx16], vals)
        for hh in prevS:
            hh.wait()
        for j, stj in enumerate((st0, st1, st2, st3)):
            pltpu.sync_copy(stj, st_hbm.at[b, hg + j])

    return scatter_k


def _attn_body(sqk_ref, sv_ref, st_ref, hk_ref, hv_ref, ht_ref,
               so_ref, slog_ref, tx_ref):
    tx_ref[0:CH] = ht_ref[0, 0]
    tx_ref[CH:CH + S] = st_ref[0, 0]

    r_i = lax.broadcasted_iota(jnp.int32, (QB, QB + CH), 0)
    c_i = lax.broadcasted_iota(jnp.int32, (QB, QB + CH), 1)
    qchunk = (r_i // CH) * CH
    band = (c_i >= qchunk) & (c_i < qchunk + 2 * CH)
    scale = D ** -0.5

    def block(i, q, kw, vw, tq, tk):
        ssq = jnp.sum(kw * kw, axis=1, keepdims=True)
        rnorm = lax.rsqrt(jnp.maximum(ssq, 1e-24))
        kn = kw * rnorm
        dots = lax.dot_general(q * scale, kn, (((1,), (1,)), ((), ())),
                               preferred_element_type=jnp.float32)
        mask = jnp.where(band & (tq != tk), 1.0, 0.0)
        p = jnp.exp(dots) * mask
        l = jnp.sum(p, axis=1, keepdims=True)
        lse = jnp.log(l)
        o = lax.dot_general(p, vw, (((1,), (0,)), ((), ())),
                            preferred_element_type=jnp.float32)
        so_ref[0, 0, pl.ds(i * QB, QB), :] = o * (1.0 / l)
        slog_ref[0, 0, pl.ds(i * QB, QB), :] = lse

    q0 = sqk_ref[0, 0, 0:QB, :]
    kw0 = jnp.concatenate([hk_ref[0, 0], sqk_ref[0, 0, 0:QB, :]], axis=0)
    vw0 = jnp.concatenate([hv_ref[0, 0], sv_ref[0, 0, 0:QB, :]], axis=0)
    tq0 = tx_ref[CH:CH + QB]
    tk0 = jnp.transpose(tx_ref[0:QB + CH], (1, 0))
    block(0, q0, kw0, vw0, tq0, tk0)

    for i in range(1, NQB):
        q = sqk_ref[0, 0, pl.ds(i * QB, QB), :]
        kw = sqk_ref[0, 0, pl.ds(i * QB - CH, QB + CH), :]
        vw = sv_ref[0, 0, pl.ds(i * QB - CH, QB + CH), :]
        tq = tx_ref[pl.ds(CH + i * QB, QB)]
        tk = jnp.transpose(tx_ref[pl.ds(i * QB, QB + CH)], (1, 0))
        block(i, q, kw, vw, tq, tk)


def _attention(sqk, sv, st2, halo_k, halo_v, halo_t2):
    return pl.pallas_call(
        _attn_body,
        grid=(B, H),
        in_specs=[
            pl.BlockSpec((1, 1, S, D), lambda b, h: (b, h, 0, 0)),
            pl.BlockSpec((1, 1, S, D), lambda b, h: (b, h, 0, 0)),
            pl.BlockSpec((1, 1, S, 1), lambda b, h: (b, h, 0, 0)),
            pl.BlockSpec((1, 1, CH, D), lambda b, h: (b, h, 0, 0)),
            pl.BlockSpec((1, 1, CH, D), lambda b, h: (b, h, 0, 0)),
            pl.BlockSpec((1, 1, CH, 1), lambda b, h: (b, h, 0, 0)),
        ],
        out_specs=[
            pl.BlockSpec((1, 1, S, D), lambda b, h: (b, h, 0, 0)),
            pl.BlockSpec((1, 1, S, 1), lambda b, h: (b, h, 0, 0)),
        ],
        out_shape=[
            jax.ShapeDtypeStruct((B, H, S, D), jnp.float32),
            jax.ShapeDtypeStruct((B, H, S, 1), jnp.float32),
        ],
        scratch_shapes=[
            pltpu.VMEM((S + CH, 1), jnp.int32),
        ],
    )(sqk, sv, st2, halo_k, halo_v, halo_t2)


def _make_gather_kernel():
    mesh = plsc.VectorSubcoreMesh(core_axis_name="c", subcore_axis_name="s")

    @functools.partial(
        pl.kernel,
        mesh=mesh,
        out_type=(
            jax.ShapeDtypeStruct((B, H, S, D), jnp.float32),
            jax.ShapeDtypeStruct((B, H, S), jnp.float32),
        ),
        scratch_types=[
            pltpu.VMEM((2, 128), jnp.int32),
            pltpu.VMEM((2, 128, D), jnp.float32),
            pltpu.VMEM((S,), jnp.float32),
            pltpu.VMEM((2, 128), jnp.float32),
            pltpu.SemaphoreType.DMA,
            pltpu.SemaphoreType.DMA,
            pltpu.SemaphoreType.DMA,
        ],
        compiler_params=pltpu.CompilerParams(needs_layout_passes=False),
    )
    def gather_k(so_hbm, slog_hbm, dest_hbm, ot_hbm, lt_hbm,
                 idx_v, rows_v, sl_v, lg_v, semI, semG, semO):
        wid = lax.axis_index("s") * 2 + lax.axis_index("c")
        for p in range(4):
            g = wid * 4 + p
            b = g // H
            h = g % H
            pltpu.sync_copy(slog_hbm.at[b, h], sl_v)
            idxh = pltpu.async_copy(dest_hbm.at[b, h, pl.ds(0, 128)],
                                    idx_v.at[0], semI)
            gh_prev = None
            out_prev = []
            for c in range(_NCHUNK):
                par = c % 2
                idxh.wait()
                for hh in out_prev:
                    hh.wait()
                gh = pltpu.async_copy(so_hbm.at[b, h].at[idx_v.at[par]],
                                      rows_v.at[par], semG)
                if gh_prev is not None:
                    for k in range(8):
                        idx16 = idx_v[1 - par, pl.ds(k * 16, 16)]
                        lg_v[1 - par, pl.ds(k * 16, 16)] = (
                            plsc.load_gather(sl_v, [idx16]))
                    gh_prev.wait()
                    ps = (c - 1) * 128
                    out_prev = [
                        pltpu.async_copy(rows_v.at[1 - par],
                                         ot_hbm.at[b, h, pl.ds(ps, 128)],
                                         semO),
                        pltpu.async_copy(lg_v.at[1 - par],
                                         lt_hbm.at[b, h, pl.ds(ps, 128)],
                                         semO),
                    ]
                    if c + 1 < _NCHUNK:
                        idxh = pltpu.async_copy(
                            dest_hbm.at[b, h, pl.ds((c + 1) * 128, 128)],
                            idx_v.at[1 - par], semI)
                elif c + 1 < _NCHUNK:
                    idxh = pltpu.async_copy(
                        dest_hbm.at[b, h, pl.ds((c + 1) * 128, 128)],
                        idx_v.at[1 - par], semI)
                gh_prev = gh
            par = (_NCHUNK - 1) % 2
            for k in range(8):
                idx16 = idx_v[par, pl.ds(k * 16, 16)]
                lg_v[par, pl.ds(k * 16, 16)] = plsc.load_gather(sl_v, [idx16])
            gh_prev.wait()
            ps = (_NCHUNK - 1) * 128
            for hh in out_prev:
                hh.wait()
            pltpu.sync_copy(rows_v.at[par], ot_hbm.at[b, h, pl.ds(ps, 128)])
            pltpu.sync_copy(lg_v.at[par], lt_hbm.at[b, h, pl.ds(ps, 128)])

    return gather_k


_TS = 512


def _combine_body(ot_ref, lt_ref, out_ref):
    lg = lt_ref[0]
    m = jnp.max(lg, axis=0, keepdims=True)
    p = jnp.exp(lg - m)
    ssum = jnp.sum(p, axis=0, keepdims=True)
    w = p / ssum
    wt = jnp.transpose(w, (1, 0))
    acc = jnp.zeros((_TS, D), jnp.float32)
    for h in range(H):
        acc = acc + ot_ref[0, h] * wt[:, h:h + 1]
    out_ref[0] = acc


def _combine(o_tok, log_tok):
    return pl.pallas_call(
        _combine_body,
        grid=(B, S // _TS),
        in_specs=[
            pl.BlockSpec((1, H, _TS, D), lambda b, t: (b, 0, t, 0)),
            pl.BlockSpec((1, H, _TS), lambda b, t: (b, 0, t)),
        ],
        out_specs=pl.BlockSpec((1, _TS, D), lambda b, t: (b, t, 0)),
        out_shape=jax.ShapeDtypeStruct((B, S, D), jnp.float32),
    )(o_tok, log_tok)


_make_scatter_kernel = functools.cache(_make_scatter_kernel)
_make_gather_kernel = functools.cache(_make_gather_kernel)


@jax.jit
def kernel(qk, v, rotations):
    rot = rotations.reshape(D, H * 16)
    dest = _hash_dest(qk, rot)
    sqk, sv, st = _make_scatter_kernel()(qk, v, dest)
    halo_k = jnp.roll(sqk[:, :, S - CH:, :], 1, axis=1)
    halo_v = jnp.roll(sv[:, :, S - CH:, :], 1, axis=1)
    halo_t = jnp.roll(st[:, :, S - CH:], 1, axis=1)
    so, slog = _attention(sqk, sv, st.reshape(B, H, S, 1),
                          halo_k, halo_v, halo_t.reshape(B, H, CH, 1))
    o_tok, log_tok = _make_gather_kernel()(so, slog.reshape(B, H, S), dest)
    return _combine(o_tok, log_tok)


# speedup vs baseline: 9.8928x; 1.0206x over previous
import functools
import jax
import jax.numpy as jnp
from jax import lax
from jax.experimental import pallas as pl
from jax.experimental.pallas import tpu as pltpu
from jax.experimental.pallas import tpu_sc as plsc

B, S, D = 16, 2048, 128
H = 8
NBK = 32
CH = 64
QB = 512
NQB = S // QB
GRP = 256
NGRP = S // GRP
SELF_VAL = -50000.0

_HIGH = lax.Precision.HIGHEST


def _hash_dest_body(qk_ref, rot_ref, dest_ref):
    qk = qk_ref[0]
    rot = rot_ref[...]
    rotated = lax.dot_general(qk, rot, (((1,), (0,)), ((), ())),
                              preferred_element_type=jnp.float32,
                              precision=lax.Precision.DEFAULT)
    ohm_bf = []
    for h in range(H):
        rh = rotated[:, h * 16:(h + 1) * 16]
        cc = jnp.concatenate([rh, -rh], axis=1)
        m = jnp.max(cc, axis=1, keepdims=True)
        ohm_bf.append((cc == m).astype(jnp.bfloat16))
    OHM = jnp.concatenate(ohm_bf, axis=1)
    a_i = lax.broadcasted_iota(jnp.int32, (H * NBK, H * NBK), 0)
    b_i = lax.broadcasted_iota(jnp.int32, (H * NBK, H * NBK), 1)
    U = ((a_i // NBK == b_i // NBK) & (a_i < b_i)).astype(jnp.bfloat16)
    prem = lax.dot_general(OHM, U, (((1,), (0,)), ((), ())),
                           preferred_element_type=jnp.float32)
    oh_f256 = jnp.where(prem == 0.0, OHM.astype(jnp.float32), 0.0)
    OH = oh_f256.astype(jnp.bfloat16)
    LT = lax.dot_general(OH, U, (((1,), (0,)), ((), ())),
                         preferred_element_type=jnp.float32)
    offs = jnp.sum(LT, axis=0, keepdims=True)
    oh_f = [oh_f256[:, h * NBK:(h + 1) * NBK] for h in range(H)]

    r_i = lax.broadcasted_iota(jnp.int32, (GRP, GRP), 0)
    c_i = lax.broadcasted_iota(jnp.int32, (GRP, GRP), 1)
    Ls = (r_i > c_i).astype(jnp.bfloat16)
    base = jnp.zeros((1, H * NBK), jnp.float32)
    parts = []
    for g in range(NGRP):
        blk = OH[g * GRP:(g + 1) * GRP]
        cumg = lax.dot_general(Ls, blk, (((1,), (0,)), ((), ())),
                               preferred_element_type=jnp.float32)
        parts.append(cumg + base)
        base = base + jnp.sum(blk.astype(jnp.float32), axis=0, keepdims=True)
    RANK = jnp.concatenate(parts, axis=0)
    destf = RANK + offs
    cols = []
    for h in range(H):
        sel = destf[:, h * NBK:(h + 1) * NBK] * oh_f[h]
        cols.append(jnp.sum(sel, axis=1, keepdims=True))
    dest = jnp.concatenate(cols, axis=1).astype(jnp.int32)
    dest_ref[0] = jnp.transpose(dest, (1, 0))


def _hash_dest(qk, rot):
    return pl.pallas_call(
        _hash_dest_body,
        grid=(B,),
        in_specs=[
            pl.BlockSpec((1, S, D), lambda b: (b, 0, 0)),
            pl.BlockSpec((D, H * 16), lambda b: (0, 0)),
        ],
        out_specs=pl.BlockSpec((1, H, S), lambda b: (b, 0, 0)),
        out_shape=jax.ShapeDtypeStruct((B, H, S), jnp.int32),
    )(qk, rot)


_NCHUNK = S // 128


def _make_scatter_kernel():
    mesh = plsc.VectorSubcoreMesh(core_axis_name="c", subcore_axis_name="s")

    @functools.partial(
        pl.kernel,
        mesh=mesh,
        out_type=(
            jax.ShapeDtypeStruct((B, H, S, D), jnp.float32),
            jax.ShapeDtypeStruct((B, H, S, D), jnp.float32),
            jax.ShapeDtypeStruct((B, H, S), jnp.int32),
        ),
        scratch_types=[
            pltpu.VMEM((8, 128), jnp.int32),
            pltpu.VMEM((2, 128, D), jnp.float32),
            pltpu.VMEM((2, 128, D), jnp.float32),
            pltpu.VMEM((S,), jnp.int32),
            pltpu.VMEM((S,), jnp.int32),
            pltpu.VMEM((S,), jnp.int32),
            pltpu.VMEM((S,), jnp.int32),
            pltpu.SemaphoreType.DMA,
            pltpu.SemaphoreType.DMA,
        ],
        compiler_params=pltpu.CompilerParams(needs_layout_passes=False),
    )
    def scatter_k(qk_hbm, v_hbm, dest_hbm, sqk_hbm, sv_hbm, st_hbm,
                  idx_v, rq_v, rv_v, st0, st1, st2, st3, semL, semS):
        wid = lax.axis_index("s") * 2 + lax.axis_index("c")
        b = wid // 2
        hg = (wid % 2) * 4

        def loads(c, par):
            cs = c * 128
            hs = [pltpu.async_copy(dest_hbm.at[b, hg + j, pl.ds(cs, 128)],
                                   idx_v.at[par * 4 + j], semL) for j in range(4)]
            hs.append(pltpu.async_copy(qk_hbm.at[b, pl.ds(cs, 128)],
                                       rq_v.at[par], semL))
            hs.append(pltpu.async_copy(v_hbm.at[b, pl.ds(cs, 128)],
                                       rv_v.at[par], semL))
            return hs

        hsL = loads(0, 0)
        prevS = []
        for c in range(_NCHUNK):
            par = c % 2
            cs = c * 128
            for hh in hsL:
                hh.wait()
            for hh in prevS:
                hh.wait()
            prevS = []
            for j in range(4):
                prevS.append(pltpu.async_copy(
                    rq_v.at[par], sqk_hbm.at[b, hg + j].at[idx_v.at[par * 4 + j]],
                    semS))
                prevS.append(pltpu.async_copy(
                    rv_v.at[par], sv_hbm.at[b, hg + j].at[idx_v.at[par * 4 + j]],
                    semS))
            if c + 1 < _NCHUNK:
                hsL = loads(c + 1, 1 - par)
            for j, stj in enumerate((st0, st1, st2, st3)):
                for k in range(8):
                    idx16 = idx_v[par * 4 + j, pl.ds(k * 16, 16)]
                    vals = (cs + k * 16) + lax.iota(jnp.int32, 16)
                    plsc.store_scatter(stj, [idx16], vals)
        for hh in prevS:
            hh.wait()
        for j, stj in enumerate((st0, st1, st2, st3)):
            pltpu.sync_copy(stj, st_hbm.at[b, hg + j])

    return scatter_k


def _attn_body(sqk_ref, sv_ref, st_ref, hk_ref, hv_ref, ht_ref,
               so_ref, slog_ref, tx_ref):
    tx_ref[0:CH] = ht_ref[0, 0]
    tx_ref[CH:CH + S] = st_ref[0, 0]

    r_i = lax.broadcasted_iota(jnp.int32, (QB, QB + CH), 0)
    c_i = lax.broadcasted_iota(jnp.int32, (QB, QB + CH), 1)
    qchunk = (r_i // CH) * CH
    band = (c_i >= qchunk) & (c_i < qchunk + 2 * CH)
    scale = D ** -0.5

    def block(i, q, kw, vw, tq, tk):
        ssq = jnp.sum(kw * kw, axis=1, keepdims=True)
        rnorm = lax.rsqrt(jnp.maximum(ssq, 1e-24))
        kn = kw * rnorm
        dots = lax.dot_general(q * scale, kn, (((1,), (1,)), ((), ())),
                               preferred_element_type=jnp.float32)
        mask = jnp.where(band & (tq != tk), 1.0, 0.0)
        p = jnp.exp(dots) * mask
        l = jnp.sum(p, axis=1, keepdims=True)
        lse = jnp.log(l)
        o = lax.dot_general(p, vw, (((1,), (0,)), ((), ())),
                            preferred_element_type=jnp.float32)
        so_ref[0, 0, pl.ds(i * QB, QB), :] = o * (1.0 / l)
        slog_ref[0, 0, pl.ds(i * QB, QB), :] = lse

    q0 = sqk_ref[0, 0, 0:QB, :]
    kw0 = jnp.concatenate([hk_ref[0, 0], sqk_ref[0, 0, 0:QB, :]], axis=0)
    vw0 = jnp.concatenate([hv_ref[0, 0], sv_ref[0, 0, 0:QB, :]], axis=0)
    tq0 = tx_ref[CH:CH + QB]
    tk0 = jnp.transpose(tx_ref[0:QB + CH], (1, 0))
    block(0, q0, kw0, vw0, tq0, tk0)

    for i in range(1, NQB):
        q = sqk_ref[0, 0, pl.ds(i * QB, QB), :]
        kw = sqk_ref[0, 0, pl.ds(i * QB - CH, QB + CH), :]
        vw = sv_ref[0, 0, pl.ds(i * QB - CH, QB + CH), :]
        tq = tx_ref[pl.ds(CH + i * QB, QB)]
        tk = jnp.transpose(tx_ref[pl.ds(i * QB, QB + CH)], (1, 0))
        block(i, q, kw, vw, tq, tk)


def _attention(sqk, sv, st2, halo_k, halo_v, halo_t2):
    return pl.pallas_call(
        _attn_body,
        grid=(B, H),
        in_specs=[
            pl.BlockSpec((1, 1, S, D), lambda b, h: (b, h, 0, 0)),
            pl.BlockSpec((1, 1, S, D), lambda b, h: (b, h, 0, 0)),
            pl.BlockSpec((1, 1, S, 1), lambda b, h: (b, h, 0, 0)),
            pl.BlockSpec((1, 1, CH, D), lambda b, h: (b, h, 0, 0)),
            pl.BlockSpec((1, 1, CH, D), lambda b, h: (b, h, 0, 0)),
            pl.BlockSpec((1, 1, CH, 1), lambda b, h: (b, h, 0, 0)),
        ],
        out_specs=[
            pl.BlockSpec((1, 1, S, D), lambda b, h: (b, h, 0, 0)),
            pl.BlockSpec((1, 1, S, 1), lambda b, h: (b, h, 0, 0)),
        ],
        out_shape=[
            jax.ShapeDtypeStruct((B, H, S, D), jnp.float32),
            jax.ShapeDtypeStruct((B, H, S, 1), jnp.float32),
        ],
        scratch_shapes=[
            pltpu.VMEM((S + CH, 1), jnp.int32),
        ],
    )(sqk, sv, st2, halo_k, halo_v, halo_t2)


def _make_gather_kernel():
    mesh = plsc.VectorSubcoreMesh(core_axis_name="c", subcore_axis_name="s")

    @functools.partial(
        pl.kernel,
        mesh=mesh,
        out_type=(
            jax.ShapeDtypeStruct((B, H, S, D), jnp.float32),
            jax.ShapeDtypeStruct((B, H, S), jnp.float32),
        ),
        scratch_types=[
            pltpu.VMEM((2, 128), jnp.int32),
            pltpu.VMEM((2, 128, D), jnp.float32),
            pltpu.VMEM((S,), jnp.float32),
            pltpu.VMEM((2, 128), jnp.float32),
            pltpu.SemaphoreType.DMA,
            pltpu.SemaphoreType.DMA,
            pltpu.SemaphoreType.DMA,
        ],
        compiler_params=pltpu.CompilerParams(needs_layout_passes=False),
    )
    def gather_k(so_hbm, slog_hbm, dest_hbm, ot_hbm, lt_hbm,
                 idx_v, rows_v, sl_v, lg_v, semI, semG, semO):
        wid = lax.axis_index("s") * 2 + lax.axis_index("c")
        for p in range(4):
            g = wid * 4 + p
            b = g // H
            h = g % H
            pltpu.sync_copy(slog_hbm.at[b, h], sl_v)
            idxh = pltpu.async_copy(dest_hbm.at[b, h, pl.ds(0, 128)],
                                    idx_v.at[0], semI)
            gh_prev = None
            out_prev = []
            for c in range(_NCHUNK):
                par = c % 2
                idxh.wait()
                for hh in out_prev:
                    hh.wait()
                gh = pltpu.async_copy(so_hbm.at[b, h].at[idx_v.at[par]],
                                      rows_v.at[par], semG)
                if gh_prev is not None:
                    for k in range(8):
                        idx16 = idx_v[1 - par, pl.ds(k * 16, 16)]
                        lg_v[1 - par, pl.ds(k * 16, 16)] = (
                            plsc.load_gather(sl_v, [idx16]))
                    gh_prev.wait()
                    ps = (c - 1) * 128
                    out_prev = [
                        pltpu.async_copy(rows_v.at[1 - par],
                                         ot_hbm.at[b, h, pl.ds(ps, 128)],
                                         semO),
                        pltpu.async_copy(lg_v.at[1 - par],
                                         lt_hbm.at[b, h, pl.ds(ps, 128)],
                                         semO),
                    ]
                    if c + 1 < _NCHUNK:
                        idxh = pltpu.async_copy(
                            dest_hbm.at[b, h, pl.ds((c + 1) * 128, 128)],
                            idx_v.at[1 - par], semI)
                elif c + 1 < _NCHUNK:
                    idxh = pltpu.async_copy(
                        dest_hbm.at[b, h, pl.ds((c + 1) * 128, 128)],
                        idx_v.at[1 - par], semI)
                gh_prev = gh
            par = (_NCHUNK - 1) % 2
            for k in range(8):
                idx16 = idx_v[par, pl.ds(k * 16, 16)]
                lg_v[par, pl.ds(k * 16, 16)] = plsc.load_gather(sl_v, [idx16])
            gh_prev.wait()
            ps = (_NCHUNK - 1) * 128
            for hh in out_prev:
                hh.wait()
            pltpu.sync_copy(rows_v.at[par], ot_hbm.at[b, h, pl.ds(ps, 128)])
            pltpu.sync_copy(lg_v.at[par], lt_hbm.at[b, h, pl.ds(ps, 128)])

    return gather_k


_TS = 512


def _combine_body(ot_ref, lt_ref, out_ref):
    lg = lt_ref[0]
    m = jnp.max(lg, axis=0, keepdims=True)
    p = jnp.exp(lg - m)
    ssum = jnp.sum(p, axis=0, keepdims=True)
    w = p / ssum
    wt = jnp.transpose(w, (1, 0))
    acc = jnp.zeros((_TS, D), jnp.float32)
    for h in range(H):
        acc = acc + ot_ref[0, h] * wt[:, h:h + 1]
    out_ref[0] = acc


def _combine(o_tok, log_tok):
    return pl.pallas_call(
        _combine_body,
        grid=(B, S // _TS),
        in_specs=[
            pl.BlockSpec((1, H, _TS, D), lambda b, t: (b, 0, t, 0)),
            pl.BlockSpec((1, H, _TS), lambda b, t: (b, 0, t)),
        ],
        out_specs=pl.BlockSpec((1, _TS, D), lambda b, t: (b, t, 0)),
        out_shape=jax.ShapeDtypeStruct((B, S, D), jnp.float32),
    )(o_tok, log_tok)


_make_scatter_kernel = functools.cache(_make_scatter_kernel)
_make_gather_kernel = functools.cache(_make_gather_kernel)


@jax.jit
def kernel(qk, v, rotations):
    rot = rotations.reshape(D, H * 16)
    dest = _hash_dest(qk, rot)
    sqk, sv, st = _make_scatter_kernel()(qk, v, dest)
    halo_k = jnp.roll(sqk[:, :, S - CH:, :], 1, axis=1)
    halo_v = jnp.roll(sv[:, :, S - CH:, :], 1, axis=1)
    halo_t = jnp.roll(st[:, :, S - CH:], 1, axis=1)
    so, slog = _attention(sqk, sv, st.reshape(B, H, S, 1),
                          halo_k, halo_v, halo_t.reshape(B, H, CH, 1))
    o_tok, log_tok = _make_gather_kernel()(so, slog.reshape(B, H, S), dest)
    return _combine(o_tok, log_tok)
